# Initial kernel scaffold; baseline (speedup 1.0000x reference)
import functools

import jax
import jax.numpy as jnp
import numpy as np
from jax import lax
from jax.experimental import pallas as pl
from jax.experimental.pallas import tpu as pltpu
from jax.experimental.pallas import tpu_sc as plsc

N = 50000
E = 800000
HID = 16
HEADS = 8
DH = 2
LAYERS = 4
SCALE = 1.0 / np.sqrt(DH)

NP = 53248
EP = 819200
NPW = NP // 32
EPW = EP // 32
TROWS = NP // 16
NR = NP // 8
ER = EP // 8
NBLK = 832
EBLK = 1024
F32 = jnp.float32


def _bd(w):
    return jnp.kron(jnp.eye(8, dtype=w.dtype), w)


def _tile8(b):
    return jnp.tile(b, 8)[None, :]


_MESH = plsc.VectorSubcoreMesh(core_axis_name="c", subcore_axis_name="s")


def _embed_body(h_hbm, e_hbm, embh_hbm, embe_hbm, he_hbm, ef0_hbm,
                idx_v, rows_v, sem):
    c = lax.axis_index("c")
    s = lax.axis_index("s")
    wid = s * 2 + c

    nbase = wid * NPW

    def nbody(i, carry):
        off = nbase + i * 128
        pltpu.sync_copy(h_hbm.at[pl.ds(off, 128)], idx_v)
        pltpu.async_copy(embh_hbm.at[idx_v], rows_v, sem).wait()
        pltpu.sync_copy(rows_v, he_hbm.at[pl.ds(off, 128)])
        return carry

    lax.fori_loop(0, NPW // 128, nbody, 0)

    ebase = wid * EPW

    def ebody(i, carry):
        off = ebase + i * 128
        pltpu.sync_copy(e_hbm.at[pl.ds(off, 128)], idx_v)
        pltpu.async_copy(embe_hbm.at[idx_v], rows_v, sem).wait()
        pltpu.sync_copy(rows_v, ef0_hbm.at[pl.ds(off, 128)])
        return carry

    lax.fori_loop(0, EPW // 128, ebody, 0)


_embed_call = functools.partial(
    pl.kernel,
    mesh=_MESH,
    out_type=[
        jax.ShapeDtypeStruct((NP, HID), F32),
        jax.ShapeDtypeStruct((EP, HID), F32),
    ],
    scratch_types=[
        pltpu.VMEM((128,), jnp.int32),
        pltpu.VMEM((128, HID), F32),
        pltpu.SemaphoreType.DMA,
    ],
)(_embed_body)


def _make_edge_pass(write_eout):
    def body(*refs):
        if write_eout:
            (q_hbm, k_hbm, v_hbm, pe_hbm, dst_hbm, src_hbm,
             eout_hbm, agg_hbm, den_hbm,
             dsti, srci, qd, ks, vs, peb, eob, wvb, wb, zb,
             spm_agg, spm_den, sem) = refs
        else:
            (q_hbm, k_hbm, v_hbm, pe_hbm, dst_hbm, src_hbm,
             agg_hbm, den_hbm,
             dsti, srci, qd, ks, vs, peb, eob, wvb, wb, zb,
             spm_agg, spm_den, sem) = refs
        c = lax.axis_index("c")
        s = lax.axis_index("s")
        wid = s * 2 + c

        def zb_body(i, carry):
            zb[i] = jnp.zeros((HID,), F32)
            return carry

        lax.fori_loop(0, 128, zb_body, 0)
        tbase = s * TROWS

        def z_body(i, carry):
            o = tbase + i * 128
            pltpu.sync_copy(zb, spm_agg.at[pl.ds(o, 128)])
            pltpu.sync_copy(zb, spm_den.at[pl.ds(o, 128)])
            return carry

        lax.fori_loop(0, TROWS // 128, z_body, 0)
        plsc.subcore_barrier()

        perm = lax.iota(jnp.int32, 16) ^ 1
        ebase = wid * EPW

        def chunk(g, carry):
            off = ebase + g * 128
            pltpu.sync_copy(dst_hbm.at[pl.ds(off, 128)], dsti)
            pltpu.sync_copy(src_hbm.at[pl.ds(off, 128)], srci)
            cp1 = pltpu.async_copy(q_hbm.at[dsti], qd, sem)
            cp2 = pltpu.async_copy(k_hbm.at[srci], ks, sem)
            cp3 = pltpu.async_copy(v_hbm.at[srci], vs, sem)
            cp4 = pltpu.async_copy(pe_hbm.at[pl.ds(off, 128)], peb, sem)
            cp1.wait()
            cp2.wait()
            cp3.wait()
            cp4.wait()

            def ed(i, carry2):
                sc = qd[i] * ks[i] * peb[i]
                if write_eout:
                    eob[i] = sc
                l2 = sc + sc.at[perm].get(mode="promise_in_bounds")
                l2 = jnp.minimum(jnp.maximum(l2, -5.0), 5.0)
                w = jnp.exp(l2)
                wb[i] = w
                wvb[i] = w * vs[i]
                return carry2

            lax.fori_loop(0, 128, ed, 0)
            if write_eout:
                pltpu.sync_copy(eob, eout_hbm.at[pl.ds(off, 128)])
            pltpu.sync_copy(wvb, spm_agg.at[dsti], add=True)
            pltpu.sync_copy(wb, spm_den.at[dsti], add=True)
            return carry

        lax.fori_loop(0, EPW // 128, chunk, 0)
        plsc.subcore_barrier()

        def co(i, carry):
            o = tbase + i * 128
            pltpu.sync_copy(spm_agg.at[pl.ds(o, 128)],
                            agg_hbm.at[c, pl.ds(o, 128)])
            pltpu.sync_copy(spm_den.at[pl.ds(o, 128)],
                            den_hbm.at[c, pl.ds(o, 128)])
            return carry

        lax.fori_loop(0, TROWS // 128, co, 0)

    outs = []
    if write_eout:
        outs.append(jax.ShapeDtypeStruct((EP, HID), F32))
    outs += [
        jax.ShapeDtypeStruct((2, NP, HID), F32),
        jax.ShapeDtypeStruct((2, NP, HID), F32),
    ]
    return functools.partial(
        pl.kernel,
        mesh=_MESH,
        out_type=outs,
        scratch_types=[
            pltpu.VMEM((128,), jnp.int32),
            pltpu.VMEM((128,), jnp.int32),
            pltpu.VMEM((128, HID), F32),
            pltpu.VMEM((128, HID), F32),
            pltpu.VMEM((128, HID), F32),
            pltpu.VMEM((128, HID), F32),
            pltpu.VMEM((128, HID), F32),
            pltpu.VMEM((128, HID), F32),
            pltpu.VMEM((128, HID), F32),
            pltpu.VMEM((128, HID), F32),
            pltpu.VMEM_SHARED((NP, HID), F32),
            pltpu.VMEM_SHARED((NP, HID), F32),
            pltpu.SemaphoreType.DMA,
        ],
    )(body)


_edge_pass = _make_edge_pass(True)
_edge_pass_last = _make_edge_pass(False)


def _ln_t(x, mavg, g, b):
    m = jnp.dot(x, mavg, preferred_element_type=F32)
    xc = x - m
    v = jnp.dot(xc * xc, mavg, preferred_element_type=F32)
    return xc * lax.rsqrt(v + 1e-5) * g + b


def _full(arr_shape):
    return pl.BlockSpec(arr_shape, lambda i: tuple(0 for _ in arr_shape))


def _h0_body(he, posr, wpos, bpos, wq, wk, wv, hf_o, q_o, k_o, v_o):
    hf = he[...] + jnp.dot(posr[...], wpos[...], preferred_element_type=F32) \
        + bpos[...]
    hf_o[...] = hf
    q_o[...] = jnp.dot(hf, wq[...], preferred_element_type=F32)
    k_o[...] = jnp.dot(hf, wk[...], preferred_element_type=F32)
    v_o[...] = jnp.dot(hf, wv[...], preferred_element_type=F32)


def _h0_call(he, posr, wpos, bpos, wq, wk, wv):
    grid = NR // NBLK
    row = pl.BlockSpec((NBLK, 128), lambda i: (i, 0))
    row64 = pl.BlockSpec((NBLK, 64), lambda i: (i, 0))
    return pl.pallas_call(
        _h0_body,
        grid=(grid,),
        in_specs=[row, row64, _full((64, 128)), _full((1, 128)),
                  _full((128, 128)), _full((128, 128)), _full((128, 128))],
        out_specs=[row, row, row, row],
        out_shape=[jax.ShapeDtypeStruct((NR, 128), F32)] * 4,
    )(he, posr, wpos, bpos, wq, wk, wv)


def _pe0_body(ef, we, pe_o):
    pe_o[...] = jnp.dot(ef[...], we[...], preferred_element_type=F32)


def _pe0_call(ef, we):
    row = pl.BlockSpec((EBLK, 128), lambda i: (i, 0))
    return pl.pallas_call(
        _pe0_body,
        grid=(ER // EBLK,),
        in_specs=[row, _full((128, 128))],
        out_specs=row,
        out_shape=jax.ShapeDtypeStruct((ER, 128), F32),
    )(ef, we)


def _make_node_body(last):
    def body(hf, agg0, agg1, den0, den1, mavg, woh, boh, g1, b1,
             w1, bb1, w2, bb2, g2, b2, *rest):
        if last:
            out_o = rest[0]
        else:
            wq, wk, wv, hf_o, q_o, k_o, v_o = rest
        attn = (agg0[...] + agg1[...]) / (den0[...] + den1[...] + 1e-9)
        h1 = _ln_t(hf[...] + jnp.dot(attn, woh[...], preferred_element_type=F32)
                   + boh[...], mavg[...], g1[...], b1[...])
        t = jnp.maximum(jnp.dot(h1, w1[...], preferred_element_type=F32)
                        + bb1[...], 0.0)
        h2 = jnp.dot(t, w2[...], preferred_element_type=F32) + bb2[...]
        hfn = _ln_t(h1 + h2, mavg[...], g2[...], b2[...])
        if last:
            i = pl.program_id(0)
            rows = i * NBLK + lax.broadcasted_iota(jnp.int32, (NBLK, 1), 0)
            valid = rows < (N // 8)
            out_o[...] = jnp.sum(jnp.where(valid, hfn, 0.0), axis=0,
                                 keepdims=True)
        else:
            hf_o[...] = hfn
            q_o[...] = jnp.dot(hfn, wq[...], preferred_element_type=F32)
            k_o[...] = jnp.dot(hfn, wk[...], preferred_element_type=F32)
            v_o[...] = jnp.dot(hfn, wv[...], preferred_element_type=F32)

    return body


def _node_call(last, hf, agg0, agg1, den0, den1, mavg, woh, boh, g1, b1,
               w1, bb1, w2, bb2, g2, b2, wq=None, wk=None, wv=None):
    grid = NR // NBLK
    row = pl.BlockSpec((NBLK, 128), lambda i: (i, 0))
    w128 = _full((128, 128))
    b128 = _full((1, 128))
    in_specs = [row] * 5 + [w128, w128, b128, b128, b128,
                            _full((128, 256)), _full((1, 256)),
                            _full((256, 128)), b128, b128, b128]
    args = [hf, agg0, agg1, den0, den1, mavg, woh, boh, g1, b1,
            w1, bb1, w2, bb2, g2, b2]
    if last:
        out_specs = pl.BlockSpec((1, 128), lambda i: (i, 0))
        out_shape = jax.ShapeDtypeStruct((grid, 128), F32)
    else:
        in_specs += [w128, w128, w128]
        args += [wq, wk, wv]
        out_specs = [row] * 4
        out_shape = [jax.ShapeDtypeStruct((NR, 128), F32)] * 4
    return pl.pallas_call(
        _make_node_body(last),
        grid=(grid,),
        in_specs=in_specs,
        out_specs=out_specs,
        out_shape=out_shape,
    )(*args)


def _edge_body(ef, eout, mavg, woe, boe, g1, b1, w1, bb1, w2, bb2, g2, b2,
               wen, ef_o, pe_o):
    e1 = _ln_t(ef[...] + jnp.dot(eout[...], woe[...], preferred_element_type=F32)
               + boe[...], mavg[...], g1[...], b1[...])
    t = jnp.maximum(jnp.dot(e1, w1[...], preferred_element_type=F32)
                    + bb1[...], 0.0)
    e2 = jnp.dot(t, w2[...], preferred_element_type=F32) + bb2[...]
    efn = _ln_t(e1 + e2, mavg[...], g2[...], b2[...])
    ef_o[...] = efn
    pe_o[...] = jnp.dot(efn, wen[...], preferred_element_type=F32)


def _edge_call(ef, eout, mavg, woe, boe, g1, b1, w1, bb1, w2, bb2, g2, b2, wen):
    row = pl.BlockSpec((EBLK, 128), lambda i: (i, 0))
    w128 = _full((128, 128))
    b128 = _full((1, 128))
    return pl.pallas_call(
        _edge_body,
        grid=(ER // EBLK,),
        in_specs=[row, row, w128, w128, b128, b128, b128,
                  _full((128, 256)), _full((1, 256)), _full((256, 128)),
                  b128, b128, b128, w128],
        out_specs=[row, row],
        out_shape=[jax.ShapeDtypeStruct((ER, 128), F32)] * 2,
    )(ef, eout, mavg, woe, boe, g1, b1, w1, bb1, w2, bb2, g2, b2, wen)


def _head_body(parts, fmat, wc1, bc1, wc2, bc2, wc3, bc3, out_o):
    t = jnp.sum(parts[...], axis=0, keepdims=True)
    g = jnp.dot(t, fmat[...], preferred_element_type=F32)
    x1 = jnp.maximum(jnp.dot(g, wc1[...], preferred_element_type=F32)
                     + bc1[...], 0.0)
    x2 = jnp.maximum(jnp.dot(x1, wc2[...], preferred_element_type=F32)
                     + bc2[...], 0.0)
    out_o[...] = jnp.dot(x2, wc3[...], preferred_element_type=F32) + bc3[...]


def _head_call(parts, fmat, wc1, bc1, wc2, bc2, wc3, bc3):
    g = parts.shape[0]
    return pl.pallas_call(
        _head_body,
        in_specs=[_full((g, 128)), _full((128, 128)), _full((128, 128)),
                  _full((1, 128)), _full((128, 128)), _full((1, 128)),
                  _full((128, 128)), _full((1, 128))],
        out_specs=_full((1, 128)),
        out_shape=jax.ShapeDtypeStruct((1, 128), F32),
    )(parts, fmat, wc1, bc1, wc2, bc2, wc3, bc3)


def kernel(h, pos_enc, e, edge_index, emb_h, emb_e, W_pos, b_pos, WQ, WK, WV,
           WE, WOh, bOh, WOe, bOe, W1h, b1h, W2h, b2h, W1e, b1e, W2e, b2e,
           ln1hg, ln1hb, ln2hg, ln2hb, ln1eg, ln1eb, ln2eg, ln2eb,
           Wc1, bc1, Wc2, bc2, Wc3, bc3):
    h_p = jnp.concatenate([h.astype(jnp.int32), jnp.zeros((NP - N,), jnp.int32)])
    e_p = jnp.concatenate([e.astype(jnp.int32), jnp.zeros((EP - E,), jnp.int32)])
    src_p = jnp.concatenate([edge_index[0].astype(jnp.int32),
                             jnp.full((EP - E,), N, jnp.int32)])
    dst_p = jnp.concatenate([edge_index[1].astype(jnp.int32),
                             jnp.full((EP - E,), N, jnp.int32)])
    posr = jnp.concatenate([pos_enc, jnp.zeros((NP - N, 8), F32)]).reshape(NR, 64)

    wpos = _bd(W_pos)
    bpos = _tile8(b_pos)
    mavg = _bd(jnp.ones((HID, HID), F32) / HID)

    WQb = [_bd(WQ[l]) for l in range(LAYERS)]
    WKb = [_bd(WK[l] * SCALE) for l in range(LAYERS)]
    WVb = [_bd(WV[l]) for l in range(LAYERS)]
    WEb = [_bd(WE[l]) for l in range(LAYERS)]
    WOhb = [_bd(WOh[l]) for l in range(LAYERS)]
    WOeb = [_bd(WOe[l]) for l in range(LAYERS)]
    W1hb = [_bd(W1h[l]) for l in range(LAYERS)]
    W2hb = [_bd(W2h[l]) for l in range(LAYERS)]
    W1eb = [_bd(W1e[l]) for l in range(LAYERS)]
    W2eb = [_bd(W2e[l]) for l in range(LAYERS)]
    bOht = [_tile8(bOh[l]) for l in range(LAYERS)]
    bOet = [_tile8(bOe[l]) for l in range(LAYERS)]
    b1ht = [_tile8(b1h[l]) for l in range(LAYERS)]
    b2ht = [_tile8(b2h[l]) for l in range(LAYERS)]
    b1et = [_tile8(b1e[l]) for l in range(LAYERS)]
    b2et = [_tile8(b2e[l]) for l in range(LAYERS)]
    g1ht = [_tile8(ln1hg[l]) for l in range(LAYERS)]
    h1bt = [_tile8(ln1hb[l]) for l in range(LAYERS)]
    g2ht = [_tile8(ln2hg[l]) for l in range(LAYERS)]
    h2bt = [_tile8(ln2hb[l]) for l in range(LAYERS)]
    g1et = [_tile8(ln1eg[l]) for l in range(LAYERS)]
    e1bt = [_tile8(ln1eb[l]) for l in range(LAYERS)]
    g2et = [_tile8(ln2eg[l]) for l in range(LAYERS)]
    e2bt = [_tile8(ln2eb[l]) for l in range(LAYERS)]

    fmat = jnp.zeros((128, 128), F32).at[:, :HID].set(
        jnp.kron(jnp.ones((8, 1), F32), jnp.eye(HID, dtype=F32)) / N)
    wc1p = jnp.zeros((128, 128), F32).at[:HID, :8].set(Wc1)
    bc1p = jnp.zeros((1, 128), F32).at[0, :8].set(bc1)
    wc2p = jnp.zeros((128, 128), F32).at[:8, :4].set(Wc2)
    bc2p = jnp.zeros((1, 128), F32).at[0, :4].set(bc2)
    wc3p = jnp.zeros((128, 128), F32).at[:4, :1].set(Wc3)
    bc3p = jnp.zeros((1, 128), F32).at[0, :1].set(bc3)

    he, ef0 = _embed_call(h_p, e_p, emb_h, emb_e)
    he = he.reshape(NR, 128)

    hf, q, k, v = _h0_call(he, posr, wpos, bpos, WQb[0], WKb[0], WVb[0])
    pe = _pe0_call(ef0.reshape(ER, 128), WEb[0])
    ef = ef0.reshape(ER, 128)

    for l in range(LAYERS):
        last = l == LAYERS - 1
        qt = q.reshape(NP, HID)
        kt = k.reshape(NP, HID)
        vt = v.reshape(NP, HID)
        pet = pe.reshape(EP, HID)
        if last:
            agg, den = _edge_pass_last(qt, kt, vt, pet, dst_p, src_p)
        else:
            eout, agg, den = _edge_pass(qt, kt, vt, pet, dst_p, src_p)
        agg0 = agg[0].reshape(NR, 128)
        agg1 = agg[1].reshape(NR, 128)
        den0 = den[0].reshape(NR, 128)
        den1 = den[1].reshape(NR, 128)
        if last:
            parts = _node_call(True, hf, agg0, agg1, den0, den1, mavg,
                               WOhb[l], bOht[l], g1ht[l], h1bt[l], W1hb[l],
                               b1ht[l], W2hb[l], b2ht[l], g2ht[l], h2bt[l])
        else:
            hf, q, k, v = _node_call(False, hf, agg0, agg1, den0, den1, mavg,
                                     WOhb[l], bOht[l], g1ht[l], h1bt[l],
                                     W1hb[l], b1ht[l], W2hb[l], b2ht[l],
                                     g2ht[l], h2bt[l],
                                     WQb[l + 1], WKb[l + 1], WVb[l + 1])
            ef, pe = _edge_call(ef, eout.reshape(ER, 128), mavg, WOeb[l],
                                bOet[l], g1et[l], e1bt[l], W1eb[l], b1et[l],
                                W2eb[l], b2et[l], g2et[l], e2bt[l], WEb[l + 1])

    out = _head_call(parts, fmat, wc1p, bc1p, wc2p, bc2p, wc3p, bc3p)
    return out[0:1, 0:1]


# speedup vs baseline: 47.0503x; 47.0503x over previous
import functools

import jax
import jax.numpy as jnp
import numpy as np
from jax import lax
from jax.experimental import pallas as pl
from jax.experimental.pallas import tpu as pltpu
from jax.experimental.pallas import tpu_sc as plsc

N = 50000
E = 800000
HID = 16
HEADS = 8
DH = 2
LAYERS = 4
SCALE = 1.0 / np.sqrt(DH)

NP = 53248
EP = 819200
NPW = NP // 32
EPW = EP // 32
TROWS = NP // 16
NR = NP // 8
ER = EP // 8
NBLK = 832
EBLK = 1024
F32 = jnp.float32


def _bd(w):
    return jnp.kron(jnp.eye(8, dtype=w.dtype), w)


def _tile8(b):
    return jnp.tile(b, 8)[None, :]


_MESH = plsc.VectorSubcoreMesh(core_axis_name="c", subcore_axis_name="s")
_SC_PARAMS = pltpu.CompilerParams(use_tc_tiling_on_sc=False)


def _embed_body(h_hbm, e_hbm, embh_hbm, embe_hbm, he_hbm, ef0_hbm,
                idx_v, rows_v, sem):
    c = lax.axis_index("c")
    s = lax.axis_index("s")
    wid = s * 2 + c

    nbase = wid * NPW

    def nbody(i, carry):
        off = nbase + i * 128
        pltpu.sync_copy(h_hbm.at[pl.ds(off, 128)], idx_v)
        pltpu.async_copy(embh_hbm.at[idx_v], rows_v, sem).wait()
        pltpu.sync_copy(rows_v, he_hbm.at[pl.ds(off, 128)])
        return carry

    lax.fori_loop(0, NPW // 128, nbody, 0)

    ebase = wid * EPW

    def ebody(i, carry):
        off = ebase + i * 128
        pltpu.sync_copy(e_hbm.at[pl.ds(off, 128)], idx_v)
        pltpu.async_copy(embe_hbm.at[idx_v], rows_v, sem).wait()
        pltpu.sync_copy(rows_v, ef0_hbm.at[pl.ds(off, 128)])
        return carry

    lax.fori_loop(0, EPW // 128, ebody, 0)


_embed_call = functools.partial(
    pl.kernel,
    mesh=_MESH,
    out_type=[
        jax.ShapeDtypeStruct((NP, HID), F32),
        jax.ShapeDtypeStruct((EP, HID), F32),
    ],
    scratch_types=[
        pltpu.VMEM((128,), jnp.int32),
        pltpu.VMEM((128, HID), F32),
        pltpu.SemaphoreType.DMA,
    ],
    compiler_params=_SC_PARAMS,
)(_embed_body)


def _make_edge_pass(write_eout):
    def body(*refs):
        if write_eout:
            (q_hbm, k_hbm, v_hbm, pe_hbm, dst_hbm, src_hbm,
             eout_hbm, agg_hbm, den_hbm,
             dsti, srci, qd, ks, vs, peb, eob, wvb, wb, zb,
             spm_agg, spm_den, sem) = refs
        else:
            (q_hbm, k_hbm, v_hbm, pe_hbm, dst_hbm, src_hbm,
             agg_hbm, den_hbm,
             dsti, srci, qd, ks, vs, peb, eob, wvb, wb, zb,
             spm_agg, spm_den, sem) = refs
        c = lax.axis_index("c")
        s = lax.axis_index("s")
        wid = s * 2 + c

        def zb_body(i, carry):
            zb[i] = jnp.zeros((HID,), F32)
            return carry

        lax.fori_loop(0, 128, zb_body, 0)
        tbase = s * TROWS

        def z_body(i, carry):
            o = tbase + i * 128
            pltpu.sync_copy(zb, spm_agg.at[pl.ds(o, 128)])
            pltpu.sync_copy(zb, spm_den.at[pl.ds(o, 128)])
            return carry

        lax.fori_loop(0, TROWS // 128, z_body, 0)
        plsc.subcore_barrier()

        perm = lax.iota(jnp.int32, 16) ^ 1
        ebase = wid * EPW

        def chunk(g, carry):
            off = ebase + g * 128
            pltpu.sync_copy(dst_hbm.at[pl.ds(off, 128)], dsti)
            pltpu.sync_copy(src_hbm.at[pl.ds(off, 128)], srci)
            cp1 = pltpu.async_copy(q_hbm.at[dsti], qd, sem)
            cp2 = pltpu.async_copy(k_hbm.at[srci], ks, sem)
            cp3 = pltpu.async_copy(v_hbm.at[srci], vs, sem)
            cp4 = pltpu.async_copy(pe_hbm.at[pl.ds(off, 128)], peb, sem)
            cp1.wait()
            cp2.wait()
            cp3.wait()
            cp4.wait()

            def ed(i, carry2):
                sc = qd[i] * ks[i] * peb[i]
                if write_eout:
                    eob[i] = sc
                l2 = sc + sc.at[perm].get(mode="promise_in_bounds")
                l2 = jnp.minimum(jnp.maximum(l2, -5.0), 5.0)
                w = jnp.exp(l2)
                wb[i] = w
                wvb[i] = w * vs[i]
                return carry2

            lax.fori_loop(0, 128, ed, 0)
            if write_eout:
                pltpu.sync_copy(eob, eout_hbm.at[pl.ds(off, 128)])
            pltpu.sync_copy(wvb, spm_agg.at[dsti], add=True)
            pltpu.sync_copy(wb, spm_den.at[dsti], add=True)
            return carry

        lax.fori_loop(0, EPW // 128, chunk, 0)
        plsc.subcore_barrier()

        def co(i, carry):
            o = tbase + i * 128
            pltpu.sync_copy(spm_agg.at[pl.ds(o, 128)],
                            agg_hbm.at[c, pl.ds(o, 128)])
            pltpu.sync_copy(spm_den.at[pl.ds(o, 128)],
                            den_hbm.at[c, pl.ds(o, 128)])
            return carry

        lax.fori_loop(0, TROWS // 128, co, 0)

    outs = []
    if write_eout:
        outs.append(jax.ShapeDtypeStruct((EP, HID), F32))
    outs += [
        jax.ShapeDtypeStruct((2, NP, HID), F32),
        jax.ShapeDtypeStruct((2, NP, HID), F32),
    ]
    return functools.partial(
        pl.kernel,
        mesh=_MESH,
        out_type=outs,
        scratch_types=[
            pltpu.VMEM((128,), jnp.int32),
            pltpu.VMEM((128,), jnp.int32),
            pltpu.VMEM((128, HID), F32),
            pltpu.VMEM((128, HID), F32),
            pltpu.VMEM((128, HID), F32),
            pltpu.VMEM((128, HID), F32),
            pltpu.VMEM((128, HID), F32),
            pltpu.VMEM((128, HID), F32),
            pltpu.VMEM((128, HID), F32),
            pltpu.VMEM((128, HID), F32),
            pltpu.VMEM_SHARED((NP, HID), F32),
            pltpu.VMEM_SHARED((NP, HID), F32),
            pltpu.SemaphoreType.DMA,
        ],
        compiler_params=_SC_PARAMS,
    )(body)


_edge_pass = _make_edge_pass(True)
_edge_pass_last = _make_edge_pass(False)


def _ln_t(x, mavg, g, b):
    m = jnp.dot(x, mavg, preferred_element_type=F32)
    xc = x - m
    v = jnp.dot(xc * xc, mavg, preferred_element_type=F32)
    return xc * lax.rsqrt(v + 1e-5) * g + b


def _full(arr_shape):
    return pl.BlockSpec(arr_shape, lambda *i: tuple(0 for _ in arr_shape))


def _h0_body(he, posr, wpos, bpos, wq, wk, wv, hf_o, q_o, k_o, v_o):
    hf = he[...] + jnp.dot(posr[...], wpos[...], preferred_element_type=F32) \
        + bpos[...]
    hf_o[...] = hf
    q_o[...] = jnp.dot(hf, wq[...], preferred_element_type=F32)
    k_o[...] = jnp.dot(hf, wk[...], preferred_element_type=F32)
    v_o[...] = jnp.dot(hf, wv[...], preferred_element_type=F32)


def _h0_call(he, posr, wpos, bpos, wq, wk, wv):
    grid = NR // NBLK
    row = pl.BlockSpec((NBLK, 128), lambda i: (i, 0))
    row64 = pl.BlockSpec((NBLK, 64), lambda i: (i, 0))
    return pl.pallas_call(
        _h0_body,
        grid=(grid,),
        in_specs=[row, row64, _full((64, 128)), _full((1, 128)),
                  _full((128, 128)), _full((128, 128)), _full((128, 128))],
        out_specs=[row, row, row, row],
        out_shape=[jax.ShapeDtypeStruct((NR, 128), F32)] * 4,
    )(he, posr, wpos, bpos, wq, wk, wv)


def _pe0_body(ef, we, pe_o):
    pe_o[...] = jnp.dot(ef[...], we[...], preferred_element_type=F32)


def _pe0_call(ef, we):
    row = pl.BlockSpec((EBLK, 128), lambda i: (i, 0))
    return pl.pallas_call(
        _pe0_body,
        grid=(ER // EBLK,),
        in_specs=[row, _full((128, 128))],
        out_specs=row,
        out_shape=jax.ShapeDtypeStruct((ER, 128), F32),
    )(ef, we)


def _make_node_body(last):
    def body(hf, agg0, agg1, den0, den1, mavg, woh, boh, g1, b1,
             w1, bb1, w2, bb2, g2, b2, *rest):
        if last:
            out_o = rest[0]
        else:
            wq, wk, wv, hf_o, q_o, k_o, v_o = rest
        attn = (agg0[...] + agg1[...]) / (den0[...] + den1[...] + 1e-9)
        h1 = _ln_t(hf[...] + jnp.dot(attn, woh[...], preferred_element_type=F32)
                   + boh[...], mavg[...], g1[...], b1[...])
        t = jnp.maximum(jnp.dot(h1, w1[...], preferred_element_type=F32)
                        + bb1[...], 0.0)
        h2 = jnp.dot(t, w2[...], preferred_element_type=F32) + bb2[...]
        hfn = _ln_t(h1 + h2, mavg[...], g2[...], b2[...])
        if last:
            i = pl.program_id(0)
            rows = i * NBLK + lax.broadcasted_iota(jnp.int32, (NBLK, 1), 0)
            valid = rows < (N // 8)
            part = jnp.sum(jnp.where(valid, hfn, 0.0), axis=0, keepdims=True)
            out_o[...] = jnp.where(
                lax.broadcasted_iota(jnp.int32, (8, 128), 0) == 0, part, 0.0)
        else:
            hf_o[...] = hfn
            q_o[...] = jnp.dot(hfn, wq[...], preferred_element_type=F32)
            k_o[...] = jnp.dot(hfn, wk[...], preferred_element_type=F32)
            v_o[...] = jnp.dot(hfn, wv[...], preferred_element_type=F32)

    return body


def _node_call(last, hf, agg0, agg1, den0, den1, mavg, woh, boh, g1, b1,
               w1, bb1, w2, bb2, g2, b2, wq=None, wk=None, wv=None):
    grid = NR // NBLK
    row = pl.BlockSpec((NBLK, 128), lambda i: (i, 0))
    w128 = _full((128, 128))
    b128 = _full((1, 128))
    in_specs = [row] * 5 + [w128, w128, b128, b128, b128,
                            _full((128, 256)), _full((1, 256)),
                            _full((256, 128)), b128, b128, b128]
    args = [hf, agg0, agg1, den0, den1, mavg, woh, boh, g1, b1,
            w1, bb1, w2, bb2, g2, b2]
    if last:
        out_specs = pl.BlockSpec((8, 128), lambda i: (i, 0))
        out_shape = jax.ShapeDtypeStruct((grid * 8, 128), F32)
    else:
        in_specs += [w128, w128, w128]
        args += [wq, wk, wv]
        out_specs = [row] * 4
        out_shape = [jax.ShapeDtypeStruct((NR, 128), F32)] * 4
    return pl.pallas_call(
        _make_node_body(last),
        grid=(grid,),
        in_specs=in_specs,
        out_specs=out_specs,
        out_shape=out_shape,
    )(*args)


def _edge_body(ef, eout, mavg, woe, boe, g1, b1, w1, bb1, w2, bb2, g2, b2,
               wen, ef_o, pe_o):
    e1 = _ln_t(ef[...] + jnp.dot(eout[...], woe[...], preferred_element_type=F32)
               + boe[...], mavg[...], g1[...], b1[...])
    t = jnp.maximum(jnp.dot(e1, w1[...], preferred_element_type=F32)
                    + bb1[...], 0.0)
    e2 = jnp.dot(t, w2[...], preferred_element_type=F32) + bb2[...]
    efn = _ln_t(e1 + e2, mavg[...], g2[...], b2[...])
    ef_o[...] = efn
    pe_o[...] = jnp.dot(efn, wen[...], preferred_element_type=F32)


def _edge_call(ef, eout, mavg, woe, boe, g1, b1, w1, bb1, w2, bb2, g2, b2, wen):
    row = pl.BlockSpec((EBLK, 128), lambda i: (i, 0))
    w128 = _full((128, 128))
    b128 = _full((1, 128))
    return pl.pallas_call(
        _edge_body,
        grid=(ER // EBLK,),
        in_specs=[row, row, w128, w128, b128, b128, b128,
                  _full((128, 256)), _full((1, 256)), _full((256, 128)),
                  b128, b128, b128, w128],
        out_specs=[row, row],
        out_shape=[jax.ShapeDtypeStruct((ER, 128), F32)] * 2,
    )(ef, eout, mavg, woe, boe, g1, b1, w1, bb1, w2, bb2, g2, b2, wen)


def _head_body(parts, fmat, wc1, bc1, wc2, bc2, wc3, bc3, out_o):
    t = jnp.sum(parts[...], axis=0, keepdims=True)
    g = jnp.dot(t, fmat[...], preferred_element_type=F32)
    x1 = jnp.maximum(jnp.dot(g, wc1[...], preferred_element_type=F32)
                     + bc1[...], 0.0)
    x2 = jnp.maximum(jnp.dot(x1, wc2[...], preferred_element_type=F32)
                     + bc2[...], 0.0)
    out_o[...] = jnp.dot(x2, wc3[...], preferred_element_type=F32) + bc3[...]


def _head_call(parts, fmat, wc1, bc1, wc2, bc2, wc3, bc3):
    g = parts.shape[0]
    return pl.pallas_call(
        _head_body,
        in_specs=[_full((g, 128)), _full((128, 128)), _full((128, 128)),
                  _full((1, 128)), _full((128, 128)), _full((1, 128)),
                  _full((128, 128)), _full((1, 128))],
        out_specs=_full((1, 128)),
        out_shape=jax.ShapeDtypeStruct((1, 128), F32),
    )(parts, fmat, wc1, bc1, wc2, bc2, wc3, bc3)


def kernel(h, pos_enc, e, edge_index, emb_h, emb_e, W_pos, b_pos, WQ, WK, WV,
           WE, WOh, bOh, WOe, bOe, W1h, b1h, W2h, b2h, W1e, b1e, W2e, b2e,
           ln1hg, ln1hb, ln2hg, ln2hb, ln1eg, ln1eb, ln2eg, ln2eb,
           Wc1, bc1, Wc2, bc2, Wc3, bc3):
    h_p = jnp.concatenate([h.astype(jnp.int32), jnp.zeros((NP - N,), jnp.int32)])
    e_p = jnp.concatenate([e.astype(jnp.int32), jnp.zeros((EP - E,), jnp.int32)])
    src_p = jnp.concatenate([edge_index[0].astype(jnp.int32),
                             jnp.full((EP - E,), N, jnp.int32)])
    dst_p = jnp.concatenate([edge_index[1].astype(jnp.int32),
                             jnp.full((EP - E,), N, jnp.int32)])
    posr = jnp.concatenate([pos_enc, jnp.zeros((NP - N, 8), F32)]).reshape(NR, 64)

    wpos = _bd(W_pos)
    bpos = _tile8(b_pos)
    mavg = _bd(jnp.ones((HID, HID), F32) / HID)

    WQb = [_bd(WQ[l]) for l in range(LAYERS)]
    WKb = [_bd(WK[l] * SCALE) for l in range(LAYERS)]
    WVb = [_bd(WV[l]) for l in range(LAYERS)]
    WEb = [_bd(WE[l]) for l in range(LAYERS)]
    WOhb = [_bd(WOh[l]) for l in range(LAYERS)]
    WOeb = [_bd(WOe[l]) for l in range(LAYERS)]
    W1hb = [_bd(W1h[l]) for l in range(LAYERS)]
    W2hb = [_bd(W2h[l]) for l in range(LAYERS)]
    W1eb = [_bd(W1e[l]) for l in range(LAYERS)]
    W2eb = [_bd(W2e[l]) for l in range(LAYERS)]
    bOht = [_tile8(bOh[l]) for l in range(LAYERS)]
    bOet = [_tile8(bOe[l]) for l in range(LAYERS)]
    b1ht = [_tile8(b1h[l]) for l in range(LAYERS)]
    b2ht = [_tile8(b2h[l]) for l in range(LAYERS)]
    b1et = [_tile8(b1e[l]) for l in range(LAYERS)]
    b2et = [_tile8(b2e[l]) for l in range(LAYERS)]
    g1ht = [_tile8(ln1hg[l]) for l in range(LAYERS)]
    h1bt = [_tile8(ln1hb[l]) for l in range(LAYERS)]
    g2ht = [_tile8(ln2hg[l]) for l in range(LAYERS)]
    h2bt = [_tile8(ln2hb[l]) for l in range(LAYERS)]
    g1et = [_tile8(ln1eg[l]) for l in range(LAYERS)]
    e1bt = [_tile8(ln1eb[l]) for l in range(LAYERS)]
    g2et = [_tile8(ln2eg[l]) for l in range(LAYERS)]
    e2bt = [_tile8(ln2eb[l]) for l in range(LAYERS)]

    fmat = jnp.zeros((128, 128), F32).at[:, :HID].set(
        jnp.kron(jnp.ones((8, 1), F32), jnp.eye(HID, dtype=F32)) / N)
    wc1p = jnp.zeros((128, 128), F32).at[:HID, :8].set(Wc1)
    bc1p = jnp.zeros((1, 128), F32).at[0, :8].set(bc1)
    wc2p = jnp.zeros((128, 128), F32).at[:8, :4].set(Wc2)
    bc2p = jnp.zeros((1, 128), F32).at[0, :4].set(bc2)
    wc3p = jnp.zeros((128, 128), F32).at[:4, :1].set(Wc3)
    bc3p = jnp.zeros((1, 128), F32).at[0, :1].set(bc3)

    he, ef0 = _embed_call(h_p, e_p, emb_h, emb_e)
    he = he.reshape(NR, 128)

    hf, q, k, v = _h0_call(he, posr, wpos, bpos, WQb[0], WKb[0], WVb[0])
    pe = _pe0_call(ef0.reshape(ER, 128), WEb[0])
    ef = ef0.reshape(ER, 128)

    for l in range(LAYERS):
        last = l == LAYERS - 1
        qt = q.reshape(NP, HID)
        kt = k.reshape(NP, HID)
        vt = v.reshape(NP, HID)
        pet = pe.reshape(EP, HID)
        if last:
            agg, den = _edge_pass_last(qt, kt, vt, pet, dst_p, src_p)
        else:
            eout, agg, den = _edge_pass(qt, kt, vt, pet, dst_p, src_p)
        agg0 = agg[0].reshape(NR, 128)
        agg1 = agg[1].reshape(NR, 128)
        den0 = den[0].reshape(NR, 128)
        den1 = den[1].reshape(NR, 128)
        if last:
            parts = _node_call(True, hf, agg0, agg1, den0, den1, mavg,
                               WOhb[l], bOht[l], g1ht[l], h1bt[l], W1hb[l],
                               b1ht[l], W2hb[l], b2ht[l], g2ht[l], h2bt[l])
        else:
            hf, q, k, v = _node_call(False, hf, agg0, agg1, den0, den1, mavg,
                                     WOhb[l], bOht[l], g1ht[l], h1bt[l],
                                     W1hb[l], b1ht[l], W2hb[l], b2ht[l],
                                     g2ht[l], h2bt[l],
                                     WQb[l + 1], WKb[l + 1], WVb[l + 1])
            ef, pe = _edge_call(ef, eout.reshape(ER, 128), mavg, WOeb[l],
                                bOet[l], g1et[l], e1bt[l], W1eb[l], b1et[l],
                                W2eb[l], b2et[l], g2et[l], e2bt[l], WEb[l + 1])

    out = _head_call(parts, fmat, wc1p, bc1p, wc2p, bc2p, wc3p, bc3p)
    return out[0:1, 0:1]


# speedup vs baseline: 135.7495x; 2.8852x over previous
import functools

import jax
import jax.numpy as jnp
import numpy as np
from jax import lax
from jax.experimental import pallas as pl
from jax.experimental.pallas import tpu as pltpu
from jax.experimental.pallas import tpu_sc as plsc

N = 50000
E = 800000
HID = 16
HEADS = 8
DH = 2
LAYERS = 4
SCALE = 1.0 / np.sqrt(DH)

NP = 50048
EP = 819200
NIR = NP // 128
EPW = EP // 32
TROWS = NP // 16
NR = NP // 8
ER = EP // 8
EBLK = 1024
F32 = jnp.float32


def _bd(w):
    return jnp.kron(jnp.eye(8, dtype=w.dtype), w)


def _tile8(b):
    return jnp.tile(b, 8)[None, :]


_MESH = plsc.VectorSubcoreMesh(core_axis_name="c", subcore_axis_name="s")
_SC_PARAMS = pltpu.CompilerParams(use_tc_tiling_on_sc=False,
                                  needs_layout_passes=False)


def _bcast_lane(vec, t):
    return vec.at[jnp.full((16,), t, jnp.int32)].get(mode="promise_in_bounds")


def _embed_body(h2_hbm, e2_hbm, embh_hbm, tabe_hbm, he_hbm, ef0_hbm, pe0_hbm,
                tabh, tabe, idxn, idxe, rowsh, re0, rp0, re1, rp1, sem, ssem0,
                ssem1):
    c = lax.axis_index("c")
    s = lax.axis_index("s")
    wid = s * 2 + c
    pltpu.sync_copy(embh_hbm, tabh)
    pltpu.sync_copy(tabe_hbm, tabe)
    pltpu.sync_copy(e2_hbm.at[pl.ds(wid * 200, 200)], idxe)
    iota16 = lax.iota(jnp.int32, 16)

    def nchunk(j, carry):
        row = j * 32 + wid

        @pl.when(row < NIR)
        def _do():
            pltpu.sync_copy(h2_hbm.at[pl.ds(row, 1)], idxn)
            for k in range(8):
                hv = idxn[0, pl.ds(k * 16, 16)]
                for t in range(16):
                    idx = _bcast_lane(hv, t) * 16 + iota16
                    rowsh[k * 16 + t] = plsc.load_gather(tabh, [idx])
            pltpu.sync_copy(rowsh, he_hbm.at[pl.ds(row * 128, 128)])

        return carry

    lax.fori_loop(0, (NIR + 31) // 32, nchunk, 0)

    nout = jnp.where(wid == 31, (E - 31 * EPW) // 256, EPW // 256)

    def echunk2(o, carry):
        for b, (re, rp, ssem) in enumerate(((re0, rp0, ssem0),
                                            (re1, rp1, ssem1))):
            j = o * 2 + b
            off = wid * EPW + j * 128

            @pl.when(o > 0)
            def _drain():
                pltpu.make_async_copy(re, ef0_hbm.at[pl.ds(off, 128)],
                                      ssem).wait()
                pltpu.make_async_copy(rp, pe0_hbm.at[pl.ds(off, 128)],
                                      ssem).wait()

            for k in range(8):
                ev = idxe[j, pl.ds(k * 16, 16)]
                for t in range(16):
                    idx = _bcast_lane(ev, t) * 16 + iota16
                    re[k * 16 + t] = plsc.load_gather(tabe, [idx])
                    rp[k * 16 + t] = plsc.load_gather(tabe, [idx + 128])
            pltpu.make_async_copy(re, ef0_hbm.at[pl.ds(off, 128)],
                                  ssem).start()
            pltpu.make_async_copy(rp, pe0_hbm.at[pl.ds(off, 128)],
                                  ssem).start()
        return carry

    lax.fori_loop(0, nout, echunk2, 0)
    for re, rp, ssem in ((re0, rp0, ssem0), (re1, rp1, ssem1)):
        pltpu.make_async_copy(re, ef0_hbm.at[pl.ds(0, 128)], ssem).wait()
        pltpu.make_async_copy(rp, pe0_hbm.at[pl.ds(0, 128)], ssem).wait()


_embed_call = functools.partial(
    pl.kernel,
    mesh=_MESH,
    out_type=[
        jax.ShapeDtypeStruct((NP, HID), F32),
        jax.ShapeDtypeStruct((EP, HID), F32),
        jax.ShapeDtypeStruct((EP, HID), F32),
    ],
    scratch_types=[
        pltpu.VMEM((100 * HID,), F32),
        pltpu.VMEM((256,), F32),
        pltpu.VMEM((1, 128), jnp.int32),
        pltpu.VMEM((200, 128), jnp.int32),
        pltpu.VMEM((128, HID), F32),
        pltpu.VMEM((128, HID), F32),
        pltpu.VMEM((128, HID), F32),
        pltpu.VMEM((128, HID), F32),
        pltpu.VMEM((128, HID), F32),
        pltpu.SemaphoreType.DMA,
        pltpu.SemaphoreType.DMA,
        pltpu.SemaphoreType.DMA,
    ],
    compiler_params=_SC_PARAMS,
)(_embed_body)


def _make_edge_pass(write_eout):
    def body(*refs):
        if write_eout:
            (q_hbm, k_hbm, v_hbm, pe_hbm, ds_hbm,
             eout_hbm, agg_hbm, den_hbm,
             idx, qd0, ks0, vs0, pb0, qd1, ks1, vs1, pb1,
             eo0, wv0, wb0, eo1,
             spm_agg, spm_den, gsem0, gsem1, ssem0, ssem1) = refs
        else:
            (q_hbm, k_hbm, v_hbm, pe_hbm, ds_hbm,
             agg_hbm, den_hbm,
             idx, qd0, ks0, vs0, pb0, qd1, ks1, vs1, pb1,
             eo0, wv0, wb0, eo1,
             spm_agg, spm_den, gsem0, gsem1, ssem0, ssem1) = refs
        c = lax.axis_index("c")
        s = lax.axis_index("s")
        wid = s * 2 + c
        gbufs = ((qd0, ks0, vs0, pb0, gsem0), (qd1, ks1, vs1, pb1, gsem1))
        sbufs = ((eo0, wv0, wb0, ssem0), (eo1, wv0, wb0, ssem1))
        zb = wv0

        def zb_body(i, carry):
            zb[i] = jnp.zeros((HID,), F32)
            return carry

        lax.fori_loop(0, 128, zb_body, 0)
        tbase = s * TROWS
        TTAIL = TROWS - (TROWS // 128) * 128

        def z_body(i, carry):
            o = tbase + i * 128
            pltpu.sync_copy(zb, spm_agg.at[pl.ds(o, 128)])
            pltpu.sync_copy(zb, spm_den.at[pl.ds(o, 128)])
            return carry

        lax.fori_loop(0, TROWS // 128, z_body, 0)
        if TTAIL:
            o = tbase + (TROWS // 128) * 128
            pltpu.sync_copy(zb.at[pl.ds(0, TTAIL)],
                            spm_agg.at[pl.ds(o, TTAIL)])
            pltpu.sync_copy(zb.at[pl.ds(0, TTAIL)],
                            spm_den.at[pl.ds(o, TTAIL)])

        plsc.subcore_barrier()

        perm = lax.iota(jnp.int32, 16) ^ 1
        ebase = wid * EPW
        gbase = wid * 200
        nchunks = jnp.where(wid == 31, (E - 31 * EPW) // 128, 200)

        def gathers(g, b):
            qd, ks, vs, pb, gsem = gbufs[b]
            return (pltpu.make_async_copy(q_hbm.at[idx.at[b, 0]], qd, gsem),
                    pltpu.make_async_copy(k_hbm.at[idx.at[b, 1]], ks, gsem),
                    pltpu.make_async_copy(v_hbm.at[idx.at[b, 1]], vs, gsem),
                    pltpu.make_async_copy(
                        pe_hbm.at[pl.ds(ebase + g * 128, 128)], pb, gsem))

        def issue(g, b):
            pltpu.sync_copy(ds_hbm.at[gbase + g], idx.at[b])
            for cp in gathers(g, b):
                cp.start()

        def process(g, b, o):
            qd, ks, vs, pb, gsem = gbufs[b]
            eob, wvb, wb, ssem = sbufs[b]
            for cp in gathers(g, b):
                cp.wait()

            @pl.when(o > 0)
            def _drain_stores():
                if write_eout:
                    pltpu.make_async_copy(
                        eob, eout_hbm.at[pl.ds(ebase, 128)], ssem).wait()

            def ed(i, carry2):
                sc = qd[i] * ks[i] * pb[i]
                if write_eout:
                    eob[i] = sc
                l2 = sc + sc.at[perm].get(mode="promise_in_bounds")
                l2 = jnp.minimum(jnp.maximum(l2, -5.0), 5.0)
                w = jnp.exp(l2)
                wb[i] = w
                wvb[i] = w * vs[i]
                return carry2

            lax.fori_loop(0, 128, ed, 0)
            if write_eout:
                pltpu.make_async_copy(
                    eob, eout_hbm.at[pl.ds(ebase + g * 128, 128)],
                    ssem).start()
            pltpu.sync_copy(wvb, spm_agg.at[idx.at[b, 0]], add=True)
            pltpu.sync_copy(wb, spm_den.at[idx.at[b, 0]], add=True)

            @pl.when(g + 2 < nchunks)
            def _issue_next():
                issue(g + 2, b)

        issue(0, 0)
        issue(1, 1)

        def chunk2(o, carry):
            process(o * 2, 0, o)
            process(o * 2 + 1, 1, o)
            return carry

        lax.fori_loop(0, nchunks // 2, chunk2, 0)
        if write_eout:
            for eob, wvb, wb, ssem in sbufs:
                pltpu.make_async_copy(
                    eob, eout_hbm.at[pl.ds(ebase, 128)], ssem).wait()
        plsc.subcore_barrier()

        def co(i, carry):
            o = tbase + i * 128
            pltpu.sync_copy(spm_agg.at[pl.ds(o, 128)],
                            agg_hbm.at[c, pl.ds(o, 128)])
            pltpu.sync_copy(spm_den.at[pl.ds(o, 128)],
                            den_hbm.at[c, pl.ds(o, 128)])
            return carry

        lax.fori_loop(0, TROWS // 128, co, 0)
        if TTAIL:
            o = tbase + (TROWS // 128) * 128
            pltpu.sync_copy(spm_agg.at[pl.ds(o, TTAIL)],
                            agg_hbm.at[c, pl.ds(o, TTAIL)])
            pltpu.sync_copy(spm_den.at[pl.ds(o, TTAIL)],
                            den_hbm.at[c, pl.ds(o, TTAIL)])

    outs = []
    if write_eout:
        outs.append(jax.ShapeDtypeStruct((EP, HID), F32))
    outs += [
        jax.ShapeDtypeStruct((2, NP, HID), F32),
        jax.ShapeDtypeStruct((2, NP, HID), F32),
    ]
    return functools.partial(
        pl.kernel,
        mesh=_MESH,
        out_type=outs,
        scratch_types=[
            pltpu.VMEM((2, 2, 128), jnp.int32),
        ] + [pltpu.VMEM((128, HID), F32)] * 12 + [
            pltpu.VMEM_SHARED((NP, HID), F32),
            pltpu.VMEM_SHARED((NP, HID), F32),
            pltpu.SemaphoreType.DMA,
            pltpu.SemaphoreType.DMA,
            pltpu.SemaphoreType.DMA,
            pltpu.SemaphoreType.DMA,
        ],
        compiler_params=_SC_PARAMS,
    )(body)


_edge_pass = _make_edge_pass(True)
_edge_pass_last = _make_edge_pass(False)


def _ln_t(x, mavg, g, b):
    m = jnp.dot(x, mavg, preferred_element_type=F32)
    xc = x - m
    v = jnp.dot(xc * xc, mavg, preferred_element_type=F32)
    return xc * lax.rsqrt(v + 1e-5) * g + b


def _full(arr_shape):
    return pl.BlockSpec(arr_shape, lambda *i: tuple(0 for _ in arr_shape))


def _h0_body(he, posr, wpos, bpos, wq, wk, wv, hf_o, q_o, k_o, v_o):
    hf = he[...] + jnp.dot(posr[...], wpos[...], preferred_element_type=F32) \
        + bpos[...]
    hf_o[...] = hf
    q_o[...] = jnp.dot(hf, wq[...], preferred_element_type=F32)
    k_o[...] = jnp.dot(hf, wk[...], preferred_element_type=F32)
    v_o[...] = jnp.dot(hf, wv[...], preferred_element_type=F32)


def _h0_call(he, posr, wpos, bpos, wq, wk, wv):
    return pl.pallas_call(
        _h0_body,
        out_shape=[jax.ShapeDtypeStruct((NR, 128), F32)] * 4,
    )(he, posr, wpos, bpos, wq, wk, wv)


def _make_node_body(last):
    def body(hf, agg0, agg1, den0, den1, mavg, woh, boh, g1, b1,
             w1, bb1, w2, bb2, g2, b2, *rest):
        if last:
            out_o = rest[0]
        else:
            wq, wk, wv, hf_o, q_o, k_o, v_o = rest
        attn = (agg0[...] + agg1[...]) / (den0[...] + den1[...] + 1e-9)
        h1 = _ln_t(hf[...] + jnp.dot(attn, woh[...], preferred_element_type=F32)
                   + boh[...], mavg[...], g1[...], b1[...])
        t = jnp.maximum(jnp.dot(h1, w1[...], preferred_element_type=F32)
                        + bb1[...], 0.0)
        h2 = jnp.dot(t, w2[...], preferred_element_type=F32) + bb2[...]
        hfn = _ln_t(h1 + h2, mavg[...], g2[...], b2[...])
        if last:
            rows = lax.broadcasted_iota(jnp.int32, (NR, 1), 0)
            valid = rows < (N // 8)
            part = jnp.sum(jnp.where(valid, hfn, 0.0), axis=0, keepdims=True)
            out_o[...] = jnp.where(
                lax.broadcasted_iota(jnp.int32, (8, 128), 0) == 0, part, 0.0)
        else:
            hf_o[...] = hfn
            q_o[...] = jnp.dot(hfn, wq[...], preferred_element_type=F32)
            k_o[...] = jnp.dot(hfn, wk[...], preferred_element_type=F32)
            v_o[...] = jnp.dot(hfn, wv[...], preferred_element_type=F32)

    return body


def _node_call(last, hf, agg0, agg1, den0, den1, mavg, woh, boh, g1, b1,
               w1, bb1, w2, bb2, g2, b2, wq=None, wk=None, wv=None):
    args = [hf, agg0, agg1, den0, den1, mavg, woh, boh, g1, b1,
            w1, bb1, w2, bb2, g2, b2]
    if last:
        out_shape = jax.ShapeDtypeStruct((8, 128), F32)
    else:
        args += [wq, wk, wv]
        out_shape = [jax.ShapeDtypeStruct((NR, 128), F32)] * 4
    return pl.pallas_call(
        _make_node_body(last),
        out_shape=out_shape,
    )(*args)


def _edge_body(ef, eout, mavg, woe, boe, g1, b1, w1, bb1, w2, bb2, g2, b2,
               wen, ef_o, pe_o):
    e1 = _ln_t(ef[...] + jnp.dot(eout[...], woe[...], preferred_element_type=F32)
               + boe[...], mavg[...], g1[...], b1[...])
    t = jnp.maximum(jnp.dot(e1, w1[...], preferred_element_type=F32)
                    + bb1[...], 0.0)
    e2 = jnp.dot(t, w2[...], preferred_element_type=F32) + bb2[...]
    efn = _ln_t(e1 + e2, mavg[...], g2[...], b2[...])
    ef_o[...] = efn
    pe_o[...] = jnp.dot(efn, wen[...], preferred_element_type=F32)


def _edge_call(ef, eout, mavg, woe, boe, g1, b1, w1, bb1, w2, bb2, g2, b2, wen):
    row = pl.BlockSpec((EBLK, 128), lambda i: (i, 0))
    w128 = _full((128, 128))
    b128 = _full((1, 128))
    return pl.pallas_call(
        _edge_body,
        grid=(ER // EBLK,),
        in_specs=[row, row, w128, w128, b128, b128, b128,
                  _full((128, 256)), _full((1, 256)), _full((256, 128)),
                  b128, b128, b128, w128],
        out_specs=[row, row],
        out_shape=[jax.ShapeDtypeStruct((ER, 128), F32)] * 2,
    )(ef, eout, mavg, woe, boe, g1, b1, w1, bb1, w2, bb2, g2, b2, wen)


def _head_body(parts, fmat, wc1, bc1, wc2, bc2, wc3, bc3, out_o):
    t = jnp.sum(parts[...], axis=0, keepdims=True)
    g = jnp.dot(t, fmat[...], preferred_element_type=F32)
    x1 = jnp.maximum(jnp.dot(g, wc1[...], preferred_element_type=F32)
                     + bc1[...], 0.0)
    x2 = jnp.maximum(jnp.dot(x1, wc2[...], preferred_element_type=F32)
                     + bc2[...], 0.0)
    out_o[...] = jnp.dot(x2, wc3[...], preferred_element_type=F32) + bc3[...]


def _head_call(parts, fmat, wc1, bc1, wc2, bc2, wc3, bc3):
    g = parts.shape[0]
    return pl.pallas_call(
        _head_body,
        in_specs=[_full((g, 128)), _full((128, 128)), _full((128, 128)),
                  _full((1, 128)), _full((128, 128)), _full((1, 128)),
                  _full((128, 128)), _full((1, 128))],
        out_specs=_full((1, 128)),
        out_shape=jax.ShapeDtypeStruct((1, 128), F32),
    )(parts, fmat, wc1, bc1, wc2, bc2, wc3, bc3)


def kernel(h, pos_enc, e, edge_index, emb_h, emb_e, W_pos, b_pos, WQ, WK, WV,
           WE, WOh, bOh, WOe, bOe, W1h, b1h, W2h, b2h, W1e, b1e, W2e, b2e,
           ln1hg, ln1hb, ln2hg, ln2hb, ln1eg, ln1eb, ln2eg, ln2eb,
           Wc1, bc1, Wc2, bc2, Wc3, bc3):
    h_p = jnp.concatenate([h.astype(jnp.int32), jnp.zeros((NP - N,), jnp.int32)])
    e_p = jnp.concatenate([e.astype(jnp.int32), jnp.zeros((EP - E,), jnp.int32)])
    src_p = jnp.concatenate([edge_index[0].astype(jnp.int32),
                             jnp.zeros((EP - E,), jnp.int32)])
    dst_p = jnp.concatenate([edge_index[1].astype(jnp.int32),
                             jnp.zeros((EP - E,), jnp.int32)])
    posr = jnp.concatenate([pos_enc, jnp.zeros((NP - N, 8), F32)]).reshape(NR, 64)
    dssrc = jnp.stack([dst_p.reshape(EP // 128, 128),
                       src_p.reshape(EP // 128, 128)], axis=1)

    wpos = _bd(W_pos)
    bpos = _tile8(b_pos)
    mavg = _bd(jnp.ones((HID, HID), F32) / HID)

    WQb = [_bd(WQ[l]) for l in range(LAYERS)]
    WKb = [_bd(WK[l] * SCALE) for l in range(LAYERS)]
    WVb = [_bd(WV[l]) for l in range(LAYERS)]
    WEb = [_bd(WE[l]) for l in range(LAYERS)]
    WOhb = [_bd(WOh[l]) for l in range(LAYERS)]
    WOeb = [_bd(WOe[l]) for l in range(LAYERS)]
    W1hb = [_bd(W1h[l]) for l in range(LAYERS)]
    W2hb = [_bd(W2h[l]) for l in range(LAYERS)]
    W1eb = [_bd(W1e[l]) for l in range(LAYERS)]
    W2eb = [_bd(W2e[l]) for l in range(LAYERS)]
    bOht = [_tile8(bOh[l]) for l in range(LAYERS)]
    bOet = [_tile8(bOe[l]) for l in range(LAYERS)]
    b1ht = [_tile8(b1h[l]) for l in range(LAYERS)]
    b2ht = [_tile8(b2h[l]) for l in range(LAYERS)]
    b1et = [_tile8(b1e[l]) for l in range(LAYERS)]
    b2et = [_tile8(b2e[l]) for l in range(LAYERS)]
    g1ht = [_tile8(ln1hg[l]) for l in range(LAYERS)]
    h1bt = [_tile8(ln1hb[l]) for l in range(LAYERS)]
    g2ht = [_tile8(ln2hg[l]) for l in range(LAYERS)]
    h2bt = [_tile8(ln2hb[l]) for l in range(LAYERS)]
    g1et = [_tile8(ln1eg[l]) for l in range(LAYERS)]
    e1bt = [_tile8(ln1eb[l]) for l in range(LAYERS)]
    g2et = [_tile8(ln2eg[l]) for l in range(LAYERS)]
    e2bt = [_tile8(ln2eb[l]) for l in range(LAYERS)]

    fmat = jnp.zeros((128, 128), F32).at[:, :HID].set(
        jnp.kron(jnp.ones((8, 1), F32), jnp.eye(HID, dtype=F32)) / N)
    wc1p = jnp.zeros((128, 128), F32).at[:HID, :8].set(Wc1)
    bc1p = jnp.zeros((1, 128), F32).at[0, :8].set(bc1)
    wc2p = jnp.zeros((128, 128), F32).at[:8, :4].set(Wc2)
    bc2p = jnp.zeros((1, 128), F32).at[0, :4].set(bc2)
    wc3p = jnp.zeros((128, 128), F32).at[:4, :1].set(Wc3)
    bc3p = jnp.zeros((1, 128), F32).at[0, :1].set(bc3)

    tabe = jnp.concatenate([emb_e.reshape(-1), (emb_e @ WE[0]).reshape(-1)])
    he, ef0, pe = _embed_call(h_p.reshape(NP // 128, 128),
                              e_p.reshape(EP // 128, 128),
                              emb_h.reshape(-1), tabe)
    he = he.reshape(NR, 128)

    hf, q, k, v = _h0_call(he, posr, wpos, bpos, WQb[0], WKb[0], WVb[0])
    ef = ef0.reshape(ER, 128)
    pe = pe.reshape(ER, 128)

    for l in range(LAYERS):
        last = l == LAYERS - 1
        qt = q.reshape(NP, HID)
        kt = k.reshape(NP, HID)
        vt = v.reshape(NP, HID)
        pet = pe.reshape(EP, HID)
        if last:
            agg, den = _edge_pass_last(qt, kt, vt, pet, dssrc)
        else:
            eout, agg, den = _edge_pass(qt, kt, vt, pet, dssrc)
        agg0 = agg[0].reshape(NR, 128)
        agg1 = agg[1].reshape(NR, 128)
        den0 = den[0].reshape(NR, 128)
        den1 = den[1].reshape(NR, 128)
        if last:
            parts = _node_call(True, hf, agg0, agg1, den0, den1, mavg,
                               WOhb[l], bOht[l], g1ht[l], h1bt[l], W1hb[l],
                               b1ht[l], W2hb[l], b2ht[l], g2ht[l], h2bt[l])
        else:
            hf, q, k, v = _node_call(False, hf, agg0, agg1, den0, den1, mavg,
                                     WOhb[l], bOht[l], g1ht[l], h1bt[l],
                                     W1hb[l], b1ht[l], W2hb[l], b2ht[l],
                                     g2ht[l], h2bt[l],
                                     WQb[l + 1], WKb[l + 1], WVb[l + 1])
            ef, pe = _edge_call(ef, eout.reshape(ER, 128), mavg, WOeb[l],
                                bOet[l], g1et[l], e1bt[l], W1eb[l], b1et[l],
                                W2eb[l], b2et[l], g2et[l], e2bt[l], WEb[l + 1])

    out = _head_call(parts, fmat, wc1p, bc1p, wc2p, bc2p, wc3p, bc3p)
    return out[0:1, 0:1]


# speedup vs baseline: 146.2805x; 1.0776x over previous
import functools

import jax
import jax.numpy as jnp
import numpy as np
from jax import lax
from jax.experimental import pallas as pl
from jax.experimental.pallas import tpu as pltpu
from jax.experimental.pallas import tpu_sc as plsc

N = 50000
E = 800000
HID = 16
HEADS = 8
DH = 2
LAYERS = 4
SCALE = 1.0 / np.sqrt(DH)

NP = 50048
EP = 819200
NIR = NP // 128
EPW = EP // 32
TROWS = NP // 16
NR = NP // 8
ER = EP // 8
EBLK = 2048
F32 = jnp.float32


def _bd(w):
    return jnp.kron(jnp.eye(8, dtype=w.dtype), w)


def _tile8(b):
    return jnp.tile(b, 8)[None, :]


_MESH = plsc.VectorSubcoreMesh(core_axis_name="c", subcore_axis_name="s")
_SC_PARAMS = pltpu.CompilerParams(use_tc_tiling_on_sc=False,
                                  needs_layout_passes=False)


def _bcast_lane(vec, t):
    return vec.at[jnp.full((16,), t, jnp.int32)].get(mode="promise_in_bounds")


def _embed_body(h2_hbm, e2_hbm, embh_hbm, tabe_hbm, he_hbm, ef0_hbm, pe0_hbm,
                tabh, tabe, idxn, idxe, rowsh, re0, rp0, re1, rp1, sem, ssem0,
                ssem1):
    c = lax.axis_index("c")
    s = lax.axis_index("s")
    wid = s * 2 + c
    pltpu.sync_copy(embh_hbm, tabh)
    pltpu.sync_copy(tabe_hbm, tabe)
    pltpu.sync_copy(e2_hbm.at[pl.ds(wid * 200, 200)], idxe)
    iota16 = lax.iota(jnp.int32, 16)

    def nchunk(j, carry):
        row = j * 32 + wid

        @pl.when(row < NIR)
        def _do():
            pltpu.sync_copy(h2_hbm.at[pl.ds(row, 1)], idxn)
            for k in range(8):
                hv = idxn[0, pl.ds(k * 16, 16)]
                for t in range(16):
                    idx = _bcast_lane(hv, t) * 16 + iota16
                    rowsh[k * 16 + t] = plsc.load_gather(tabh, [idx])
            pltpu.sync_copy(rowsh, he_hbm.at[pl.ds(row * 128, 128)])

        return carry

    lax.fori_loop(0, (NIR + 31) // 32, nchunk, 0)

    nout = jnp.where(wid == 31, (E - 31 * EPW) // 256, EPW // 256)

    def echunk2(o, carry):
        for b, (re, rp, ssem) in enumerate(((re0, rp0, ssem0),
                                            (re1, rp1, ssem1))):
            j = o * 2 + b
            off = wid * EPW + j * 128

            @pl.when(o > 0)
            def _drain():
                pltpu.make_async_copy(re, ef0_hbm.at[pl.ds(off, 128)],
                                      ssem).wait()
                pltpu.make_async_copy(rp, pe0_hbm.at[pl.ds(off, 128)],
                                      ssem).wait()

            for k in range(8):
                ev = idxe[j, pl.ds(k * 16, 16)]
                for t in range(16):
                    idx = _bcast_lane(ev, t) * 16 + iota16
                    re[k * 16 + t] = plsc.load_gather(tabe, [idx])
                    rp[k * 16 + t] = plsc.load_gather(tabe, [idx + 128])
            pltpu.make_async_copy(re, ef0_hbm.at[pl.ds(off, 128)],
                                  ssem).start()
            pltpu.make_async_copy(rp, pe0_hbm.at[pl.ds(off, 128)],
                                  ssem).start()
        return carry

    lax.fori_loop(0, nout, echunk2, 0)
    for re, rp, ssem in ((re0, rp0, ssem0), (re1, rp1, ssem1)):
        pltpu.make_async_copy(re, ef0_hbm.at[pl.ds(0, 128)], ssem).wait()
        pltpu.make_async_copy(rp, pe0_hbm.at[pl.ds(0, 128)], ssem).wait()


_embed_call = functools.partial(
    pl.kernel,
    mesh=_MESH,
    out_type=[
        jax.ShapeDtypeStruct((NP, HID), F32),
        jax.ShapeDtypeStruct((EP, HID), F32),
        jax.ShapeDtypeStruct((EP, HID), F32),
    ],
    scratch_types=[
        pltpu.VMEM((100 * HID,), F32),
        pltpu.VMEM((256,), F32),
        pltpu.VMEM((1, 128), jnp.int32),
        pltpu.VMEM((200, 128), jnp.int32),
        pltpu.VMEM((128, HID), F32),
        pltpu.VMEM((128, HID), F32),
        pltpu.VMEM((128, HID), F32),
        pltpu.VMEM((128, HID), F32),
        pltpu.VMEM((128, HID), F32),
        pltpu.SemaphoreType.DMA,
        pltpu.SemaphoreType.DMA,
        pltpu.SemaphoreType.DMA,
    ],
    compiler_params=_SC_PARAMS,
)(_embed_body)


def _make_edge_pass(write_eout):
    def body(*refs):
        if write_eout:
            (q_hbm, k_hbm, v_hbm, pe_hbm, ds_hbm,
             eout_hbm, agg_hbm, den_hbm,
             idx, qd0, ks0, vs0, pb0, qd1, ks1, vs1, pb1,
             eo0, wv0, wb0, eo1,
             spm_agg, spm_den, gsem0, gsem1, ssem0, ssem1) = refs
        else:
            (q_hbm, k_hbm, v_hbm, pe_hbm, ds_hbm,
             agg_hbm, den_hbm,
             idx, qd0, ks0, vs0, pb0, qd1, ks1, vs1, pb1,
             eo0, wv0, wb0, eo1,
             spm_agg, spm_den, gsem0, gsem1, ssem0, ssem1) = refs
        c = lax.axis_index("c")
        s = lax.axis_index("s")
        wid = s * 2 + c
        gbufs = ((qd0, ks0, vs0, pb0, gsem0), (qd1, ks1, vs1, pb1, gsem1))
        sbufs = ((eo0, wv0, wb0, ssem0), (eo1, wv0, wb0, ssem1))
        zb = wv0

        def zb_body(i, carry):
            zb[i] = jnp.zeros((HID,), F32)
            return carry

        lax.fori_loop(0, 128, zb_body, 0)
        tbase = s * TROWS
        TTAIL = TROWS - (TROWS // 128) * 128

        def z_body(i, carry):
            o = tbase + i * 128
            pltpu.sync_copy(zb, spm_agg.at[pl.ds(o, 128)])
            pltpu.sync_copy(zb, spm_den.at[pl.ds(o, 128)])
            return carry

        lax.fori_loop(0, TROWS // 128, z_body, 0)
        if TTAIL:
            o = tbase + (TROWS // 128) * 128
            pltpu.sync_copy(zb.at[pl.ds(0, TTAIL)],
                            spm_agg.at[pl.ds(o, TTAIL)])
            pltpu.sync_copy(zb.at[pl.ds(0, TTAIL)],
                            spm_den.at[pl.ds(o, TTAIL)])

        plsc.subcore_barrier()

        perm = lax.iota(jnp.int32, 16) ^ 1
        ebase = wid * EPW
        gbase = wid * 200
        nchunks = jnp.where(wid == 31, (E - 31 * EPW) // 128, 200)

        def gathers(g, b):
            qd, ks, vs, pb, gsem = gbufs[b]
            return (pltpu.make_async_copy(q_hbm.at[idx.at[b, 0]], qd, gsem),
                    pltpu.make_async_copy(k_hbm.at[idx.at[b, 1]], ks, gsem),
                    pltpu.make_async_copy(v_hbm.at[idx.at[b, 1]], vs, gsem),
                    pltpu.make_async_copy(
                        pe_hbm.at[pl.ds(ebase + g * 128, 128)], pb, gsem))

        def issue(g, b):
            pltpu.sync_copy(ds_hbm.at[gbase + g], idx.at[b])
            for cp in gathers(g, b):
                cp.start()

        def process(g, b, o):
            qd, ks, vs, pb, gsem = gbufs[b]
            eob, wvb, wb, ssem = sbufs[b]
            for cp in gathers(g, b):
                cp.wait()

            @pl.when(o > 0)
            def _drain_stores():
                if write_eout:
                    pltpu.make_async_copy(
                        eob, eout_hbm.at[pl.ds(ebase, 128)], ssem).wait()

            @plsc.parallel_loop(0, 128, unroll=4)
            def ed(i):
                sc = qd[i] * ks[i] * pb[i]
                if write_eout:
                    eob[i] = sc
                l2 = sc + sc.at[perm].get(mode="promise_in_bounds")
                l2 = jnp.minimum(jnp.maximum(l2, -5.0), 5.0)
                w = jnp.exp(l2)
                wb[i] = w
                wvb[i] = w * vs[i]
            if write_eout:
                pltpu.make_async_copy(
                    eob, eout_hbm.at[pl.ds(ebase + g * 128, 128)],
                    ssem).start()
            pltpu.sync_copy(wvb, spm_agg.at[idx.at[b, 0]], add=True)
            pltpu.sync_copy(wb, spm_den.at[idx.at[b, 0]], add=True)

            @pl.when(g + 2 < nchunks)
            def _issue_next():
                issue(g + 2, b)

        issue(0, 0)
        issue(1, 1)

        def chunk2(o, carry):
            process(o * 2, 0, o)
            process(o * 2 + 1, 1, o)
            return carry

        lax.fori_loop(0, nchunks // 2, chunk2, 0)
        if write_eout:
            for eob, wvb, wb, ssem in sbufs:
                pltpu.make_async_copy(
                    eob, eout_hbm.at[pl.ds(ebase, 128)], ssem).wait()
        plsc.subcore_barrier()

        def co(i, carry):
            o = tbase + i * 128
            pltpu.sync_copy(spm_agg.at[pl.ds(o, 128)],
                            agg_hbm.at[c, pl.ds(o, 128)])
            pltpu.sync_copy(spm_den.at[pl.ds(o, 128)],
                            den_hbm.at[c, pl.ds(o, 128)])
            return carry

        lax.fori_loop(0, TROWS // 128, co, 0)
        if TTAIL:
            o = tbase + (TROWS // 128) * 128
            pltpu.sync_copy(spm_agg.at[pl.ds(o, TTAIL)],
                            agg_hbm.at[c, pl.ds(o, TTAIL)])
            pltpu.sync_copy(spm_den.at[pl.ds(o, TTAIL)],
                            den_hbm.at[c, pl.ds(o, TTAIL)])

    outs = []
    if write_eout:
        outs.append(jax.ShapeDtypeStruct((EP, HID), F32))
    outs += [
        jax.ShapeDtypeStruct((2, NP, HID), F32),
        jax.ShapeDtypeStruct((2, NP, HID), F32),
    ]
    return functools.partial(
        pl.kernel,
        mesh=_MESH,
        out_type=outs,
        scratch_types=[
            pltpu.VMEM((2, 2, 128), jnp.int32),
        ] + [pltpu.VMEM((128, HID), F32)] * 12 + [
            pltpu.VMEM_SHARED((NP, HID), F32),
            pltpu.VMEM_SHARED((NP, HID), F32),
            pltpu.SemaphoreType.DMA,
            pltpu.SemaphoreType.DMA,
            pltpu.SemaphoreType.DMA,
            pltpu.SemaphoreType.DMA,
        ],
        compiler_params=_SC_PARAMS,
    )(body)


_edge_pass = _make_edge_pass(True)
_edge_pass_last = _make_edge_pass(False)


def _ln_t(x, mavg, g, b):
    m = jnp.dot(x, mavg, preferred_element_type=F32)
    xc = x - m
    v = jnp.dot(xc * xc, mavg, preferred_element_type=F32)
    return xc * lax.rsqrt(v + 1e-5) * g + b


def _full(arr_shape):
    return pl.BlockSpec(arr_shape, lambda *i: tuple(0 for _ in arr_shape))


def _h0_body(he, posr, wpos, bpos, wq, wk, wv, hf_o, q_o, k_o, v_o):
    hf = he[...] + jnp.dot(posr[...], wpos[...], preferred_element_type=F32) \
        + bpos[...]
    hf_o[...] = hf
    q_o[...] = jnp.dot(hf, wq[...], preferred_element_type=F32)
    k_o[...] = jnp.dot(hf, wk[...], preferred_element_type=F32)
    v_o[...] = jnp.dot(hf, wv[...], preferred_element_type=F32)


def _h0_call(he, posr, wpos, bpos, wq, wk, wv):
    return pl.pallas_call(
        _h0_body,
        out_shape=[jax.ShapeDtypeStruct((NR, 128), F32)] * 4,
    )(he, posr, wpos, bpos, wq, wk, wv)


def _make_node_body(last):
    def body(hf, agg0, agg1, den0, den1, mavg, woh, boh, g1, b1,
             w1, bb1, w2, bb2, g2, b2, *rest):
        if last:
            out_o = rest[0]
        else:
            wq, wk, wv, hf_o, q_o, k_o, v_o = rest
        attn = (agg0[...] + agg1[...]) / (den0[...] + den1[...] + 1e-9)
        h1 = _ln_t(hf[...] + jnp.dot(attn, woh[...], preferred_element_type=F32)
                   + boh[...], mavg[...], g1[...], b1[...])
        t = jnp.maximum(jnp.dot(h1, w1[...], preferred_element_type=F32)
                        + bb1[...], 0.0)
        h2 = jnp.dot(t, w2[...], preferred_element_type=F32) + bb2[...]
        hfn = _ln_t(h1 + h2, mavg[...], g2[...], b2[...])
        if last:
            rows = lax.broadcasted_iota(jnp.int32, (NR, 1), 0)
            valid = rows < (N // 8)
            part = jnp.sum(jnp.where(valid, hfn, 0.0), axis=0, keepdims=True)
            out_o[...] = jnp.where(
                lax.broadcasted_iota(jnp.int32, (8, 128), 0) == 0, part, 0.0)
        else:
            hf_o[...] = hfn
            q_o[...] = jnp.dot(hfn, wq[...], preferred_element_type=F32)
            k_o[...] = jnp.dot(hfn, wk[...], preferred_element_type=F32)
            v_o[...] = jnp.dot(hfn, wv[...], preferred_element_type=F32)

    return body


def _node_call(last, hf, agg0, agg1, den0, den1, mavg, woh, boh, g1, b1,
               w1, bb1, w2, bb2, g2, b2, wq=None, wk=None, wv=None):
    args = [hf, agg0, agg1, den0, den1, mavg, woh, boh, g1, b1,
            w1, bb1, w2, bb2, g2, b2]
    if last:
        out_shape = jax.ShapeDtypeStruct((8, 128), F32)
    else:
        args += [wq, wk, wv]
        out_shape = [jax.ShapeDtypeStruct((NR, 128), F32)] * 4
    return pl.pallas_call(
        _make_node_body(last),
        out_shape=out_shape,
    )(*args)


def _edge_body(ef, eout, mavg, woe, boe, g1, b1, w1, bb1, w2, bb2, g2, b2,
               wen, ef_o, pe_o):
    e1 = _ln_t(ef[...] + jnp.dot(eout[...], woe[...], preferred_element_type=F32)
               + boe[...], mavg[...], g1[...], b1[...])
    t = jnp.maximum(jnp.dot(e1, w1[...], preferred_element_type=F32)
                    + bb1[...], 0.0)
    e2 = jnp.dot(t, w2[...], preferred_element_type=F32) + bb2[...]
    efn = _ln_t(e1 + e2, mavg[...], g2[...], b2[...])
    ef_o[...] = efn
    pe_o[...] = jnp.dot(efn, wen[...], preferred_element_type=F32)


def _edge_call(ef, eout, mavg, woe, boe, g1, b1, w1, bb1, w2, bb2, g2, b2, wen):
    row = pl.BlockSpec((EBLK, 128), lambda i: (i, 0))
    w128 = _full((128, 128))
    b128 = _full((1, 128))
    return pl.pallas_call(
        _edge_body,
        grid=(ER // EBLK,),
        in_specs=[row, row, w128, w128, b128, b128, b128,
                  _full((128, 256)), _full((1, 256)), _full((256, 128)),
                  b128, b128, b128, w128],
        out_specs=[row, row],
        out_shape=[jax.ShapeDtypeStruct((ER, 128), F32)] * 2,
    )(ef, eout, mavg, woe, boe, g1, b1, w1, bb1, w2, bb2, g2, b2, wen)


def _head_body(parts, fmat, wc1, bc1, wc2, bc2, wc3, bc3, out_o):
    t = jnp.sum(parts[...], axis=0, keepdims=True)
    g = jnp.dot(t, fmat[...], preferred_element_type=F32)
    x1 = jnp.maximum(jnp.dot(g, wc1[...], preferred_element_type=F32)
                     + bc1[...], 0.0)
    x2 = jnp.maximum(jnp.dot(x1, wc2[...], preferred_element_type=F32)
                     + bc2[...], 0.0)
    out_o[...] = jnp.dot(x2, wc3[...], preferred_element_type=F32) + bc3[...]


def _head_call(parts, fmat, wc1, bc1, wc2, bc2, wc3, bc3):
    g = parts.shape[0]
    return pl.pallas_call(
        _head_body,
        in_specs=[_full((g, 128)), _full((128, 128)), _full((128, 128)),
                  _full((1, 128)), _full((128, 128)), _full((1, 128)),
                  _full((128, 128)), _full((1, 128))],
        out_specs=_full((1, 128)),
        out_shape=jax.ShapeDtypeStruct((1, 128), F32),
    )(parts, fmat, wc1, bc1, wc2, bc2, wc3, bc3)


def kernel(h, pos_enc, e, edge_index, emb_h, emb_e, W_pos, b_pos, WQ, WK, WV,
           WE, WOh, bOh, WOe, bOe, W1h, b1h, W2h, b2h, W1e, b1e, W2e, b2e,
           ln1hg, ln1hb, ln2hg, ln2hb, ln1eg, ln1eb, ln2eg, ln2eb,
           Wc1, bc1, Wc2, bc2, Wc3, bc3):
    h_p = jnp.concatenate([h.astype(jnp.int32), jnp.zeros((NP - N,), jnp.int32)])
    e_p = jnp.concatenate([e.astype(jnp.int32), jnp.zeros((EP - E,), jnp.int32)])
    src_p = jnp.concatenate([edge_index[0].astype(jnp.int32),
                             jnp.zeros((EP - E,), jnp.int32)])
    dst_p = jnp.concatenate([edge_index[1].astype(jnp.int32),
                             jnp.zeros((EP - E,), jnp.int32)])
    posr = jnp.concatenate([pos_enc, jnp.zeros((NP - N, 8), F32)]).reshape(NR, 64)
    dssrc = jnp.stack([dst_p.reshape(EP // 128, 128),
                       src_p.reshape(EP // 128, 128)], axis=1)

    wpos = _bd(W_pos)
    bpos = _tile8(b_pos)
    mavg = _bd(jnp.ones((HID, HID), F32) / HID)

    WQb = [_bd(WQ[l]) for l in range(LAYERS)]
    WKb = [_bd(WK[l] * SCALE) for l in range(LAYERS)]
    WVb = [_bd(WV[l]) for l in range(LAYERS)]
    WEb = [_bd(WE[l]) for l in range(LAYERS)]
    WOhb = [_bd(WOh[l]) for l in range(LAYERS)]
    WOeb = [_bd(WOe[l]) for l in range(LAYERS)]
    W1hb = [_bd(W1h[l]) for l in range(LAYERS)]
    W2hb = [_bd(W2h[l]) for l in range(LAYERS)]
    W1eb = [_bd(W1e[l]) for l in range(LAYERS)]
    W2eb = [_bd(W2e[l]) for l in range(LAYERS)]
    bOht = [_tile8(bOh[l]) for l in range(LAYERS)]
    bOet = [_tile8(bOe[l]) for l in range(LAYERS)]
    b1ht = [_tile8(b1h[l]) for l in range(LAYERS)]
    b2ht = [_tile8(b2h[l]) for l in range(LAYERS)]
    b1et = [_tile8(b1e[l]) for l in range(LAYERS)]
    b2et = [_tile8(b2e[l]) for l in range(LAYERS)]
    g1ht = [_tile8(ln1hg[l]) for l in range(LAYERS)]
    h1bt = [_tile8(ln1hb[l]) for l in range(LAYERS)]
    g2ht = [_tile8(ln2hg[l]) for l in range(LAYERS)]
    h2bt = [_tile8(ln2hb[l]) for l in range(LAYERS)]
    g1et = [_tile8(ln1eg[l]) for l in range(LAYERS)]
    e1bt = [_tile8(ln1eb[l]) for l in range(LAYERS)]
    g2et = [_tile8(ln2eg[l]) for l in range(LAYERS)]
    e2bt = [_tile8(ln2eb[l]) for l in range(LAYERS)]

    fmat = jnp.zeros((128, 128), F32).at[:, :HID].set(
        jnp.kron(jnp.ones((8, 1), F32), jnp.eye(HID, dtype=F32)) / N)
    wc1p = jnp.zeros((128, 128), F32).at[:HID, :8].set(Wc1)
    bc1p = jnp.zeros((1, 128), F32).at[0, :8].set(bc1)
    wc2p = jnp.zeros((128, 128), F32).at[:8, :4].set(Wc2)
    bc2p = jnp.zeros((1, 128), F32).at[0, :4].set(bc2)
    wc3p = jnp.zeros((128, 128), F32).at[:4, :1].set(Wc3)
    bc3p = jnp.zeros((1, 128), F32).at[0, :1].set(bc3)

    tabe = jnp.concatenate([emb_e.reshape(-1), (emb_e @ WE[0]).reshape(-1)])
    he, ef0, pe = _embed_call(h_p.reshape(NP // 128, 128),
                              e_p.reshape(EP // 128, 128),
                              emb_h.reshape(-1), tabe)
    he = he.reshape(NR, 128)

    hf, q, k, v = _h0_call(he, posr, wpos, bpos, WQb[0], WKb[0], WVb[0])
    ef = ef0.reshape(ER, 128)
    pe = pe.reshape(ER, 128)

    for l in range(LAYERS):
        last = l == LAYERS - 1
        qt = q.reshape(NP, HID)
        kt = k.reshape(NP, HID)
        vt = v.reshape(NP, HID)
        pet = pe.reshape(EP, HID)
        if last:
            agg, den = _edge_pass_last(qt, kt, vt, pet, dssrc)
        else:
            eout, agg, den = _edge_pass(qt, kt, vt, pet, dssrc)
        agg0 = agg[0].reshape(NR, 128)
        agg1 = agg[1].reshape(NR, 128)
        den0 = den[0].reshape(NR, 128)
        den1 = den[1].reshape(NR, 128)
        if last:
            parts = _node_call(True, hf, agg0, agg1, den0, den1, mavg,
                               WOhb[l], bOht[l], g1ht[l], h1bt[l], W1hb[l],
                               b1ht[l], W2hb[l], b2ht[l], g2ht[l], h2bt[l])
        else:
            hf, q, k, v = _node_call(False, hf, agg0, agg1, den0, den1, mavg,
                                     WOhb[l], bOht[l], g1ht[l], h1bt[l],
                                     W1hb[l], b1ht[l], W2hb[l], b2ht[l],
                                     g2ht[l], h2bt[l],
                                     WQb[l + 1], WKb[l + 1], WVb[l + 1])
            ef, pe = _edge_call(ef, eout.reshape(ER, 128), mavg, WOeb[l],
                                bOet[l], g1et[l], e1bt[l], W1eb[l], b1et[l],
                                W2eb[l], b2et[l], g2et[l], e2bt[l], WEb[l + 1])

    out = _head_call(parts, fmat, wc1p, bc1p, wc2p, bc2p, wc3p, bc3p)
    return out[0:1, 0:1]


# speedup vs baseline: 176.3016x; 1.2052x over previous
import functools

import jax
import jax.numpy as jnp
import numpy as np
from jax import lax
from jax.experimental import pallas as pl
from jax.experimental.pallas import tpu as pltpu
from jax.experimental.pallas import tpu_sc as plsc

N = 50000
E = 800000
HID = 16
HEADS = 8
DH = 2
LAYERS = 4
SCALE = 1.0 / np.sqrt(DH)

NP = 50048
EP = 819200
NIR = NP // 128
EPW = EP // 32
TROWS = NP // 16
NR = NP // 8
ER = EP // 8
EBLK = 2048
F32 = jnp.float32


def _bd(w):
    return jnp.kron(jnp.eye(8, dtype=w.dtype), w)


def _tile8(b):
    return jnp.tile(b, 8)[None, :]


_MESH = plsc.VectorSubcoreMesh(core_axis_name="c", subcore_axis_name="s")
_SC_PARAMS = pltpu.CompilerParams(use_tc_tiling_on_sc=False,
                                  needs_layout_passes=False)


def _bcast_lane(vec, t):
    return vec.at[jnp.full((16,), t, jnp.int32)].get(mode="promise_in_bounds")


def _embed_body(h2_hbm, e2_hbm, embh_hbm, tabe_hbm, he_hbm, ef0_hbm, pe0_hbm,
                tabh, tabe, idxn, idxe, rowsh, re0, rp0, re1, rp1, sem, ssem0,
                ssem1):
    c = lax.axis_index("c")
    s = lax.axis_index("s")
    wid = s * 2 + c
    pltpu.sync_copy(embh_hbm, tabh)
    pltpu.sync_copy(tabe_hbm, tabe)
    pltpu.sync_copy(e2_hbm.at[pl.ds(wid * 200, 200)], idxe)
    iota16 = lax.iota(jnp.int32, 16)

    def nchunk(j, carry):
        row = j * 32 + wid

        @pl.when(row < NIR)
        def _do():
            pltpu.sync_copy(h2_hbm.at[pl.ds(row, 1)], idxn)
            for k in range(8):
                hv = idxn[0, pl.ds(k * 16, 16)]
                for t in range(16):
                    idx = _bcast_lane(hv, t) * 16 + iota16
                    rowsh[k * 16 + t] = plsc.load_gather(tabh, [idx])
            pltpu.sync_copy(rowsh, he_hbm.at[pl.ds(row * 128, 128)])

        return carry

    lax.fori_loop(0, (NIR + 31) // 32, nchunk, 0)

    nout = jnp.where(wid == 31, (E - 31 * EPW) // 256, EPW // 256)

    def echunk2(o, carry):
        for b, (re, rp, ssem) in enumerate(((re0, rp0, ssem0),
                                            (re1, rp1, ssem1))):
            j = o * 2 + b
            off = wid * EPW + j * 128

            @pl.when(o > 0)
            def _drain():
                pltpu.make_async_copy(re, ef0_hbm.at[pl.ds(off, 128)],
                                      ssem).wait()
                pltpu.make_async_copy(rp, pe0_hbm.at[pl.ds(off, 128)],
                                      ssem).wait()

            for k in range(8):
                ev = idxe[j, pl.ds(k * 16, 16)]
                for t in range(16):
                    idx = _bcast_lane(ev, t) * 16 + iota16
                    re[k * 16 + t] = plsc.load_gather(tabe, [idx])
                    rp[k * 16 + t] = plsc.load_gather(tabe, [idx + 128])
            pltpu.make_async_copy(re, ef0_hbm.at[pl.ds(off, 128)],
                                  ssem).start()
            pltpu.make_async_copy(rp, pe0_hbm.at[pl.ds(off, 128)],
                                  ssem).start()
        return carry

    lax.fori_loop(0, nout, echunk2, 0)
    for re, rp, ssem in ((re0, rp0, ssem0), (re1, rp1, ssem1)):
        pltpu.make_async_copy(re, ef0_hbm.at[pl.ds(0, 128)], ssem).wait()
        pltpu.make_async_copy(rp, pe0_hbm.at[pl.ds(0, 128)], ssem).wait()


_embed_call = functools.partial(
    pl.kernel,
    mesh=_MESH,
    out_type=[
        jax.ShapeDtypeStruct((NP, HID), F32),
        jax.ShapeDtypeStruct((EP, HID), F32),
        jax.ShapeDtypeStruct((EP, HID), F32),
    ],
    scratch_types=[
        pltpu.VMEM((100 * HID,), F32),
        pltpu.VMEM((256,), F32),
        pltpu.VMEM((1, 128), jnp.int32),
        pltpu.VMEM((200, 128), jnp.int32),
        pltpu.VMEM((128, HID), F32),
        pltpu.VMEM((128, HID), F32),
        pltpu.VMEM((128, HID), F32),
        pltpu.VMEM((128, HID), F32),
        pltpu.VMEM((128, HID), F32),
        pltpu.SemaphoreType.DMA,
        pltpu.SemaphoreType.DMA,
        pltpu.SemaphoreType.DMA,
    ],
    compiler_params=_SC_PARAMS,
)(_embed_body)


def _make_edge_pass(write_eout):
    def body(*refs):
        if write_eout:
            (q_hbm, k_hbm, v_hbm, pe_hbm, ds_hbm,
             eout_hbm, agg_hbm,
             idx, qd0, ks0, vs0, pb0, qd1, ks1, vs1, pb1,
             eo0, eo1, ws0, ws1,
             spm, gsem0, gsem1, ssem0, ssem1) = refs
        else:
            (q_hbm, k_hbm, v_hbm, pe_hbm, ds_hbm,
             agg_hbm,
             idx, qd0, ks0, vs0, pb0, qd1, ks1, vs1, pb1,
             eo0, eo1, ws0, ws1,
             spm, gsem0, gsem1, ssem0, ssem1) = refs
        c = lax.axis_index("c")
        s = lax.axis_index("s")
        wid = s * 2 + c
        gbufs = ((qd0, ks0, vs0, pb0, gsem0), (qd1, ks1, vs1, pb1, gsem1))
        sbufs = ((eo0, ws0, ssem0), (eo1, ws1, ssem1))

        def zb_body(i, carry):
            ws0[i, pl.ds(0, HID)] = jnp.zeros((HID,), F32)
            ws0[i, pl.ds(HID, HID)] = jnp.zeros((HID,), F32)
            return carry

        lax.fori_loop(0, 128, zb_body, 0)
        tbase = s * TROWS
        TTAIL = TROWS - (TROWS // 128) * 128

        def z_body(i, carry):
            o = tbase + i * 128
            pltpu.sync_copy(ws0, spm.at[pl.ds(o, 128)])
            return carry

        lax.fori_loop(0, TROWS // 128, z_body, 0)
        if TTAIL:
            o = tbase + (TROWS // 128) * 128
            pltpu.sync_copy(ws0.at[pl.ds(0, TTAIL)],
                            spm.at[pl.ds(o, TTAIL)])

        plsc.subcore_barrier()

        perm = lax.iota(jnp.int32, 16) ^ 1
        ebase = wid * EPW
        gbase = wid * 200
        nchunks = jnp.where(wid == 31, (E - 31 * EPW) // 128, 200)

        def gathers(g, b):
            qd, ks, vs, pb, gsem = gbufs[b]
            slot = lax.rem(g, 4)
            return (pltpu.make_async_copy(q_hbm.at[idx.at[slot, 0]], qd, gsem),
                    pltpu.make_async_copy(k_hbm.at[idx.at[slot, 1]], ks, gsem),
                    pltpu.make_async_copy(v_hbm.at[idx.at[slot, 1]], vs, gsem),
                    pltpu.make_async_copy(
                        pe_hbm.at[pl.ds(ebase + g * 128, 128)], pb, gsem))

        def issue(g, b):
            pltpu.sync_copy(ds_hbm.at[gbase + g], idx.at[lax.rem(g, 4)])
            for cp in gathers(g, b):
                cp.start()

        def process(g, b, o):
            qd, ks, vs, pb, gsem = gbufs[b]
            eob, wsb, ssem = sbufs[b]
            for cp in gathers(g, b):
                cp.wait()

            @pl.when(o > 0)
            def _drain_stores():
                if write_eout:
                    pltpu.make_async_copy(
                        eob, eout_hbm.at[pl.ds(ebase, 128)], ssem).wait()

            @plsc.parallel_loop(0, 128, unroll=4)
            def ed(i):
                sc = qd[i] * ks[i] * pb[i]
                if write_eout:
                    eob[i] = sc
                l2 = sc + sc.at[perm].get(mode="promise_in_bounds")
                l2 = jnp.minimum(jnp.maximum(l2, -5.0), 5.0)
                w = jnp.exp(l2)
                wsb[i, pl.ds(0, HID)] = w * vs[i]
                wsb[i, pl.ds(HID, HID)] = w
            if write_eout:
                pltpu.make_async_copy(
                    eob, eout_hbm.at[pl.ds(ebase + g * 128, 128)],
                    ssem).start()
            pltpu.sync_copy(wsb, spm.at[idx.at[lax.rem(g, 4), 0]], add=True)

            @pl.when(g + 2 < nchunks)
            def _issue_next():
                issue(g + 2, b)

        issue(0, 0)
        issue(1, 1)

        def chunk2(o, carry):
            process(o * 2, 0, o)
            process(o * 2 + 1, 1, o)
            return carry

        lax.fori_loop(0, nchunks // 2, chunk2, 0)
        if write_eout:
            for eob, wsb, ssem in sbufs:
                pltpu.make_async_copy(
                    eob, eout_hbm.at[pl.ds(ebase, 128)], ssem).wait()
        plsc.subcore_barrier()

        def co(i, carry):
            o = tbase + i * 128
            pltpu.sync_copy(spm.at[pl.ds(o, 128)],
                            agg_hbm.at[c, pl.ds(o, 128)])
            return carry

        lax.fori_loop(0, TROWS // 128, co, 0)
        if TTAIL:
            o = tbase + (TROWS // 128) * 128
            pltpu.sync_copy(spm.at[pl.ds(o, TTAIL)],
                            agg_hbm.at[c, pl.ds(o, TTAIL)])

    outs = []
    if write_eout:
        outs.append(jax.ShapeDtypeStruct((EP, HID), F32))
    outs.append(jax.ShapeDtypeStruct((2, NP, 2 * HID), F32))
    return functools.partial(
        pl.kernel,
        mesh=_MESH,
        out_type=outs,
        scratch_types=[
            pltpu.VMEM((4, 2, 128), jnp.int32),
        ] + [pltpu.VMEM((128, HID), F32)] * 10 + [
            pltpu.VMEM((128, 2 * HID), F32),
            pltpu.VMEM((128, 2 * HID), F32),
            pltpu.VMEM_SHARED((NP, 2 * HID), F32),
            pltpu.SemaphoreType.DMA,
            pltpu.SemaphoreType.DMA,
            pltpu.SemaphoreType.DMA,
            pltpu.SemaphoreType.DMA,
        ],
        compiler_params=_SC_PARAMS,
    )(body)


_edge_pass = _make_edge_pass(True)
_edge_pass_last = _make_edge_pass(False)


def _ln_t(x, mavg, g, b):
    m = jnp.dot(x, mavg, preferred_element_type=F32)
    xc = x - m
    v = jnp.dot(xc * xc, mavg, preferred_element_type=F32)
    return xc * lax.rsqrt(v + 1e-5) * g + b


def _full(arr_shape):
    return pl.BlockSpec(arr_shape, lambda *i: tuple(0 for _ in arr_shape))


def _h0_body(he, posr, wpos, bpos, wq, wk, wv, hf_o, q_o, k_o, v_o):
    hf = he[...] + jnp.dot(posr[...], wpos[...], preferred_element_type=F32) \
        + bpos[...]
    hf_o[...] = hf
    q_o[...] = jnp.dot(hf, wq[...], preferred_element_type=F32)
    k_o[...] = jnp.dot(hf, wk[...], preferred_element_type=F32)
    v_o[...] = jnp.dot(hf, wv[...], preferred_element_type=F32)


def _h0_call(he, posr, wpos, bpos, wq, wk, wv):
    return pl.pallas_call(
        _h0_body,
        out_shape=[jax.ShapeDtypeStruct((NR, 128), F32)] * 4,
    )(he, posr, wpos, bpos, wq, wk, wv)


def _make_node_body(last):
    def body(hf, agg0, agg1, sa, sb, mavg, woh, boh, g1, b1,
             w1, bb1, w2, bb2, g2, b2, *rest):
        if last:
            out_o = rest[0]
        else:
            wq, wk, wv, hf_o, q_o, k_o, v_o = rest
        acc = agg0[...] + agg1[...]
        attn = jnp.dot(acc, sa[...], preferred_element_type=F32) / (
            jnp.dot(acc, sb[...], preferred_element_type=F32) + 1e-9)
        h1 = _ln_t(hf[...] + jnp.dot(attn, woh[...], preferred_element_type=F32)
                   + boh[...], mavg[...], g1[...], b1[...])
        t = jnp.maximum(jnp.dot(h1, w1[...], preferred_element_type=F32)
                        + bb1[...], 0.0)
        h2 = jnp.dot(t, w2[...], preferred_element_type=F32) + bb2[...]
        hfn = _ln_t(h1 + h2, mavg[...], g2[...], b2[...])
        if last:
            rows = lax.broadcasted_iota(jnp.int32, (NR, 1), 0)
            valid = rows < (N // 8)
            part = jnp.sum(jnp.where(valid, hfn, 0.0), axis=0, keepdims=True)
            out_o[...] = jnp.where(
                lax.broadcasted_iota(jnp.int32, (8, 128), 0) == 0, part, 0.0)
        else:
            hf_o[...] = hfn
            q_o[...] = jnp.dot(hfn, wq[...], preferred_element_type=F32)
            k_o[...] = jnp.dot(hfn, wk[...], preferred_element_type=F32)
            v_o[...] = jnp.dot(hfn, wv[...], preferred_element_type=F32)

    return body


def _node_call(last, hf, agg0, agg1, sa, sb, mavg, woh, boh, g1, b1,
               w1, bb1, w2, bb2, g2, b2, wq=None, wk=None, wv=None):
    args = [hf, agg0, agg1, sa, sb, mavg, woh, boh, g1, b1,
            w1, bb1, w2, bb2, g2, b2]
    if last:
        out_shape = jax.ShapeDtypeStruct((8, 128), F32)
    else:
        args += [wq, wk, wv]
        out_shape = [jax.ShapeDtypeStruct((NR, 128), F32)] * 4
    return pl.pallas_call(
        _make_node_body(last),
        out_shape=out_shape,
    )(*args)


def _edge_body(ef, eout, mavg, woe, boe, g1, b1, w1, bb1, w2, bb2, g2, b2,
               wen, ef_o, pe_o):
    e1 = _ln_t(ef[...] + jnp.dot(eout[...], woe[...], preferred_element_type=F32)
               + boe[...], mavg[...], g1[...], b1[...])
    t = jnp.maximum(jnp.dot(e1, w1[...], preferred_element_type=F32)
                    + bb1[...], 0.0)
    e2 = jnp.dot(t, w2[...], preferred_element_type=F32) + bb2[...]
    efn = _ln_t(e1 + e2, mavg[...], g2[...], b2[...])
    ef_o[...] = efn
    pe_o[...] = jnp.dot(efn, wen[...], preferred_element_type=F32)


def _edge_call(ef, eout, mavg, woe, boe, g1, b1, w1, bb1, w2, bb2, g2, b2, wen):
    row = pl.BlockSpec((EBLK, 128), lambda i: (i, 0))
    w128 = _full((128, 128))
    b128 = _full((1, 128))
    return pl.pallas_call(
        _edge_body,
        grid=(ER // EBLK,),
        in_specs=[row, row, w128, w128, b128, b128, b128,
                  _full((128, 256)), _full((1, 256)), _full((256, 128)),
                  b128, b128, b128, w128],
        out_specs=[row, row],
        out_shape=[jax.ShapeDtypeStruct((ER, 128), F32)] * 2,
    )(ef, eout, mavg, woe, boe, g1, b1, w1, bb1, w2, bb2, g2, b2, wen)


def _head_body(parts, fmat, wc1, bc1, wc2, bc2, wc3, bc3, out_o):
    t = jnp.sum(parts[...], axis=0, keepdims=True)
    g = jnp.dot(t, fmat[...], preferred_element_type=F32)
    x1 = jnp.maximum(jnp.dot(g, wc1[...], preferred_element_type=F32)
                     + bc1[...], 0.0)
    x2 = jnp.maximum(jnp.dot(x1, wc2[...], preferred_element_type=F32)
                     + bc2[...], 0.0)
    out_o[...] = jnp.dot(x2, wc3[...], preferred_element_type=F32) + bc3[...]


def _head_call(parts, fmat, wc1, bc1, wc2, bc2, wc3, bc3):
    g = parts.shape[0]
    return pl.pallas_call(
        _head_body,
        in_specs=[_full((g, 128)), _full((128, 128)), _full((128, 128)),
                  _full((1, 128)), _full((128, 128)), _full((1, 128)),
                  _full((128, 128)), _full((1, 128))],
        out_specs=_full((1, 128)),
        out_shape=jax.ShapeDtypeStruct((1, 128), F32),
    )(parts, fmat, wc1, bc1, wc2, bc2, wc3, bc3)


def kernel(h, pos_enc, e, edge_index, emb_h, emb_e, W_pos, b_pos, WQ, WK, WV,
           WE, WOh, bOh, WOe, bOe, W1h, b1h, W2h, b2h, W1e, b1e, W2e, b2e,
           ln1hg, ln1hb, ln2hg, ln2hb, ln1eg, ln1eb, ln2eg, ln2eb,
           Wc1, bc1, Wc2, bc2, Wc3, bc3):
    h_p = jnp.concatenate([h.astype(jnp.int32), jnp.zeros((NP - N,), jnp.int32)])
    e_p = jnp.concatenate([e.astype(jnp.int32), jnp.zeros((EP - E,), jnp.int32)])
    src_p = jnp.concatenate([edge_index[0].astype(jnp.int32),
                             jnp.zeros((EP - E,), jnp.int32)])
    dst_p = jnp.concatenate([edge_index[1].astype(jnp.int32),
                             jnp.zeros((EP - E,), jnp.int32)])
    posr = jnp.concatenate([pos_enc, jnp.zeros((NP - N, 8), F32)]).reshape(NR, 64)
    dssrc = jnp.stack([dst_p.reshape(EP // 128, 128),
                       src_p.reshape(EP // 128, 128)], axis=1)

    wpos = _bd(W_pos)
    bpos = _tile8(b_pos)
    mavg = _bd(jnp.ones((HID, HID), F32) / HID)
    eye16 = jnp.eye(HID, dtype=F32)
    z16 = jnp.zeros((HID, HID), F32)
    selwv = jnp.kron(jnp.eye(8, dtype=F32),
                     jnp.concatenate([eye16, z16], axis=0))
    selw = jnp.kron(jnp.eye(8, dtype=F32),
                    jnp.concatenate([z16, eye16], axis=0))

    WQb = [_bd(WQ[l]) for l in range(LAYERS)]
    WKb = [_bd(WK[l] * SCALE) for l in range(LAYERS)]
    WVb = [_bd(WV[l]) for l in range(LAYERS)]
    WEb = [_bd(WE[l]) for l in range(LAYERS)]
    WOhb = [_bd(WOh[l]) for l in range(LAYERS)]
    WOeb = [_bd(WOe[l]) for l in range(LAYERS)]
    W1hb = [_bd(W1h[l]) for l in range(LAYERS)]
    W2hb = [_bd(W2h[l]) for l in range(LAYERS)]
    W1eb = [_bd(W1e[l]) for l in range(LAYERS)]
    W2eb = [_bd(W2e[l]) for l in range(LAYERS)]
    bOht = [_tile8(bOh[l]) for l in range(LAYERS)]
    bOet = [_tile8(bOe[l]) for l in range(LAYERS)]
    b1ht = [_tile8(b1h[l]) for l in range(LAYERS)]
    b2ht = [_tile8(b2h[l]) for l in range(LAYERS)]
    b1et = [_tile8(b1e[l]) for l in range(LAYERS)]
    b2et = [_tile8(b2e[l]) for l in range(LAYERS)]
    g1ht = [_tile8(ln1hg[l]) for l in range(LAYERS)]
    h1bt = [_tile8(ln1hb[l]) for l in range(LAYERS)]
    g2ht = [_tile8(ln2hg[l]) for l in range(LAYERS)]
    h2bt = [_tile8(ln2hb[l]) for l in range(LAYERS)]
    g1et = [_tile8(ln1eg[l]) for l in range(LAYERS)]
    e1bt = [_tile8(ln1eb[l]) for l in range(LAYERS)]
    g2et = [_tile8(ln2eg[l]) for l in range(LAYERS)]
    e2bt = [_tile8(ln2eb[l]) for l in range(LAYERS)]

    fmat = jnp.zeros((128, 128), F32).at[:, :HID].set(
        jnp.kron(jnp.ones((8, 1), F32), jnp.eye(HID, dtype=F32)) / N)
    wc1p = jnp.zeros((128, 128), F32).at[:HID, :8].set(Wc1)
    bc1p = jnp.zeros((1, 128), F32).at[0, :8].set(bc1)
    wc2p = jnp.zeros((128, 128), F32).at[:8, :4].set(Wc2)
    bc2p = jnp.zeros((1, 128), F32).at[0, :4].set(bc2)
    wc3p = jnp.zeros((128, 128), F32).at[:4, :1].set(Wc3)
    bc3p = jnp.zeros((1, 128), F32).at[0, :1].set(bc3)

    tabe = jnp.concatenate([emb_e.reshape(-1), (emb_e @ WE[0]).reshape(-1)])
    he, ef0, pe = _embed_call(h_p.reshape(NP // 128, 128),
                              e_p.reshape(EP // 128, 128),
                              emb_h.reshape(-1), tabe)
    he = he.reshape(NR, 128)

    hf, q, k, v = _h0_call(he, posr, wpos, bpos, WQb[0], WKb[0], WVb[0])
    ef = ef0.reshape(ER, 128)
    pe = pe.reshape(ER, 128)

    for l in range(LAYERS):
        last = l == LAYERS - 1
        qt = q.reshape(NP, HID)
        kt = k.reshape(NP, HID)
        vt = v.reshape(NP, HID)
        pet = pe.reshape(EP, HID)
        if last:
            (agg,) = _edge_pass_last(qt, kt, vt, pet, dssrc)
        else:
            eout, agg = _edge_pass(qt, kt, vt, pet, dssrc)
        agg0 = agg[0].reshape(NR, 256)
        agg1 = agg[1].reshape(NR, 256)
        if last:
            parts = _node_call(True, hf, agg0, agg1, selwv, selw, mavg,
                               WOhb[l], bOht[l], g1ht[l], h1bt[l], W1hb[l],
                               b1ht[l], W2hb[l], b2ht[l], g2ht[l], h2bt[l])
        else:
            hf, q, k, v = _node_call(False, hf, agg0, agg1, selwv, selw, mavg,
                                     WOhb[l], bOht[l], g1ht[l], h1bt[l],
                                     W1hb[l], b1ht[l], W2hb[l], b2ht[l],
                                     g2ht[l], h2bt[l],
                                     WQb[l + 1], WKb[l + 1], WVb[l + 1])
            ef, pe = _edge_call(ef, eout.reshape(ER, 128), mavg, WOeb[l],
                                bOet[l], g1et[l], e1bt[l], W1eb[l], b1et[l],
                                W2eb[l], b2et[l], g2et[l], e2bt[l], WEb[l + 1])

    out = _head_call(parts, fmat, wc1p, bc1p, wc2p, bc2p, wc3p, bc3p)
    return out[0:1, 0:1]
